# speedup vs baseline: 1.0520x; 1.0520x over previous
import jax
import jax.numpy as jnp
from jax.experimental import pallas as pl

N_NODES = 10000
D = 128
BN = 1000


def _dense1(x, W, b, relu):
    def body(x_ref, w_ref, b_ref, o_ref):
        y = jnp.dot(x_ref[...], w_ref[...], preferred_element_type=jnp.float32)
        y = y + b_ref[...]
        if relu:
            y = jnp.maximum(y, 0.0)
        o_ref[...] = y

    n = x.shape[0]
    return pl.pallas_call(
        body,
        grid=(n // BN,),
        in_specs=[
            pl.BlockSpec((BN, D), lambda i: (i, 0)),
            pl.BlockSpec((D, D), lambda i: (0, 0)),
            pl.BlockSpec((1, D), lambda i: (0, 0)),
        ],
        out_specs=pl.BlockSpec((BN, D), lambda i: (i, 0)),
        out_shape=jax.ShapeDtypeStruct((n, D), jnp.float32),
    )(x, W, b.reshape(1, D))


def _dense2(x, Wa, n_agg, Wb, b, relu):
    def body(x_ref, wa_ref, n_ref, wb_ref, b_ref, o_ref):
        y = jnp.dot(x_ref[...], wa_ref[...], preferred_element_type=jnp.float32)
        y = y + jnp.dot(n_ref[...], wb_ref[...], preferred_element_type=jnp.float32)
        y = y + b_ref[...]
        if relu:
            y = jnp.maximum(y, 0.0)
        o_ref[...] = y

    n = x.shape[0]
    return pl.pallas_call(
        body,
        grid=(n // BN,),
        in_specs=[
            pl.BlockSpec((BN, D), lambda i: (i, 0)),
            pl.BlockSpec((D, D), lambda i: (0, 0)),
            pl.BlockSpec((BN, D), lambda i: (i, 0)),
            pl.BlockSpec((D, D), lambda i: (0, 0)),
            pl.BlockSpec((1, D), lambda i: (0, 0)),
        ],
        out_specs=pl.BlockSpec((BN, D), lambda i: (i, 0)),
        out_shape=jax.ShapeDtypeStruct((n, D), jnp.float32),
    )(x, Wa, n_agg, Wb, b.reshape(1, D))


def _segmax(h_pool, src, dst):
    msgs = jnp.take(h_pool, src, axis=0)
    neigh = jax.ops.segment_max(msgs, dst, num_segments=N_NODES)
    return jnp.where(jnp.isfinite(neigh), neigh, 0.0)


def _dot(h, src, dst):
    return jnp.sum(jnp.take(h, src, axis=0) * jnp.take(h, dst, axis=0),
                   axis=-1, keepdims=True)


def kernel(x, edge_index, neg_edge_index, Wp1, bp1, Ws1, Wn1, b1,
           Wp2, bp2, Ws2, Wn2, b2):
    src, dst = edge_index[0], edge_index[1]
    p1 = _dense1(x, Wp1, bp1, relu=True)
    n1 = _segmax(p1, src, dst)
    h1 = _dense2(x, Ws1, n1, Wn1, b1, relu=True)
    p2 = _dense1(h1, Wp2, bp2, relu=True)
    n2 = _segmax(p2, src, dst)
    h2 = _dense2(h1, Ws2, n2, Wn2, b2, relu=False)
    pos = _dot(h2, src, dst)
    neg = _dot(h2, neg_edge_index[0], neg_edge_index[1])
    return (pos, neg)


# speedup vs baseline: 1.7895x; 1.7011x over previous
import functools

import jax
import jax.numpy as jnp
from jax import lax
from jax.experimental import pallas as pl
from jax.experimental.pallas import tpu as pltpu
from jax.experimental.pallas import tpu_sc as plsc

N_NODES = 10000
E = 320000
D = 128
BN = 1000

NC = 2
NS = 16
NW = NC * NS
RPT = 313
LAST_ROWS = N_NODES - (NW - 1) * RPT
CHUNK = 2560
NCHUNKS = E // CHUNK
NGROUPS = CHUNK // 16
K = 256


def _dense1(x, W, b, relu):
    def body(x_ref, w_ref, b_ref, o_ref):
        y = jnp.dot(x_ref[...], w_ref[...], preferred_element_type=jnp.float32)
        y = y + b_ref[...]
        if relu:
            y = jnp.maximum(y, 0.0)
        o_ref[...] = y

    n = x.shape[0]
    return pl.pallas_call(
        body,
        grid=(n // BN,),
        in_specs=[
            pl.BlockSpec((BN, D), lambda i: (i, 0)),
            pl.BlockSpec((D, D), lambda i: (0, 0)),
            pl.BlockSpec((1, D), lambda i: (0, 0)),
        ],
        out_specs=pl.BlockSpec((BN, D), lambda i: (i, 0)),
        out_shape=jax.ShapeDtypeStruct((n, D), jnp.float32),
    )(x, W, b.reshape(1, D))


def _dense2(x, Wa, n_agg, Wb, b, relu):
    def body(x_ref, wa_ref, n_ref, wb_ref, b_ref, o_ref):
        y = jnp.dot(x_ref[...], wa_ref[...], preferred_element_type=jnp.float32)
        y = y + jnp.dot(n_ref[...], wb_ref[...], preferred_element_type=jnp.float32)
        y = y + b_ref[...]
        if relu:
            y = jnp.maximum(y, 0.0)
        o_ref[...] = y

    n = x.shape[0]
    return pl.pallas_call(
        body,
        grid=(n // BN,),
        in_specs=[
            pl.BlockSpec((BN, D), lambda i: (i, 0)),
            pl.BlockSpec((D, D), lambda i: (0, 0)),
            pl.BlockSpec((BN, D), lambda i: (i, 0)),
            pl.BlockSpec((D, D), lambda i: (0, 0)),
            pl.BlockSpec((1, D), lambda i: (0, 0)),
        ],
        out_specs=pl.BlockSpec((BN, D), lambda i: (i, 0)),
        out_shape=jax.ShapeDtypeStruct((n, D), jnp.float32),
    )(x, Wa, n_agg, Wb, b.reshape(1, D))


def _segmax_sc(p, src, dst):
    mesh = plsc.VectorSubcoreMesh(core_axis_name="c", subcore_axis_name="s")

    @functools.partial(
        pl.kernel,
        out_type=jax.ShapeDtypeStruct((N_NODES * D,), jnp.float32),
        mesh=mesh,
        compiler_params=pltpu.CompilerParams(needs_layout_passes=False),
        scratch_types=[
            pltpu.VMEM((CHUNK,), jnp.int32),
            pltpu.VMEM((CHUNK,), jnp.int32),
            pltpu.VMEM((K + 16,), jnp.int32),
            pltpu.VMEM((K + 16,), jnp.int32),
            pltpu.VMEM((K,), jnp.int32),
            pltpu.VMEM((K, D), jnp.float32),
            pltpu.VMEM(((RPT + 1) * D,), jnp.float32),
            pltpu.SemaphoreType.DMA,
        ],
    )
    def seg_kernel(p_hbm, src_hbm, dst_hbm, out_hbm,
                   srcv, dstv, cbs, cbl, gidx, rows, acc, sem):
        wid = lax.axis_index("s") * NC + lax.axis_index("c")
        lo = wid * RPT

        def zero_body(i, carry):
            acc[pl.ds(i * 16, 16)] = jnp.zeros((16,), jnp.float32)
            return carry

        lax.fori_loop(0, (RPT + 1) * D // 16, zero_body, 0)

        def flush():
            for g in range(K // 16):
                gidx[pl.ds(g * 16, 16)] = cbs[pl.ds(g * 16, 16)]
            pltpu.async_copy(p_hbm.at[gidx], rows, sem).wait()

            def per_edge(k, carry):
                ld = cbl[pl.ds(k, 16)][0]
                base = ld * D
                for c8 in range(D // 16):
                    sl = pl.ds(base + c8 * 16, 16)
                    acc[sl] = jnp.maximum(acc[sl], rows[k, pl.ds(c8 * 16, 16)])
                return carry

            lax.fori_loop(0, K, per_edge, 0)

        def chunk_body(ci, nacc):
            ebase = ci * CHUNK
            pltpu.sync_copy(src_hbm.at[pl.ds(ebase, CHUNK)], srcv)
            pltpu.sync_copy(dst_hbm.at[pl.ds(ebase, CHUNK)], dstv)

            def group_body(g, nacc):
                off = g * 16
                d16 = dstv[pl.ds(off, 16)]
                s16 = srcv[pl.ds(off, 16)]
                l16 = d16 - lo
                m = (l16 >= 0) & (l16 < RPT)
                nh = plsc.all_reduce_population_count(m)[0]

                @pl.when(nh > 0)
                def _():
                    plsc.store_compressed(cbs.at[pl.ds(nacc, 16)], s16, mask=m)
                    plsc.store_compressed(cbl.at[pl.ds(nacc, 16)], l16, mask=m)

                nacc = nacc + nh
                full = nacc >= K

                @pl.when(full)
                def _():
                    flush()
                    cbs[pl.ds(0, 16)] = cbs[pl.ds(K, 16)]
                    cbl[pl.ds(0, 16)] = cbl[pl.ds(K, 16)]

                return jnp.where(full, nacc - K, nacc)

            return lax.fori_loop(0, NGROUPS, group_body, nacc)

        nacc = lax.fori_loop(0, NCHUNKS, chunk_body, jnp.int32(0))

        lane = lax.iota(jnp.int32, 16)

        def pad_body(g, carry):
            off = g * 16
            idx = lane + off
            keep = idx < nacc
            cbs[pl.ds(off, 16)] = jnp.where(keep, cbs[pl.ds(off, 16)], 0)
            cbl[pl.ds(off, 16)] = jnp.where(keep, cbl[pl.ds(off, 16)], RPT)
            return carry

        lax.fori_loop(0, K // 16, pad_body, 0)

        @pl.when(nacc > 0)
        def _():
            flush()

        @pl.when(wid < NW - 1)
        def _():
            pltpu.sync_copy(acc.at[pl.ds(0, RPT * D)],
                            out_hbm.at[pl.ds(lo * D, RPT * D)])

        @pl.when(wid == NW - 1)
        def _():
            pltpu.sync_copy(acc.at[pl.ds(0, LAST_ROWS * D)],
                            out_hbm.at[pl.ds(lo * D, LAST_ROWS * D)])

    return seg_kernel(p, src, dst)


EPT = 2 * E // NW
KP = 400
NPCHUNKS = EPT // KP


def _edge_dots_sc(h, src_all, dst_all):
    mesh = plsc.VectorSubcoreMesh(core_axis_name="c", subcore_axis_name="s")

    @functools.partial(
        pl.kernel,
        out_type=jax.ShapeDtypeStruct((2 * E,), jnp.float32),
        mesh=mesh,
        compiler_params=pltpu.CompilerParams(needs_layout_passes=False),
        scratch_types=[
            pltpu.VMEM((KP,), jnp.int32),
            pltpu.VMEM((KP,), jnp.int32),
            pltpu.VMEM((KP, D), jnp.float32),
            pltpu.VMEM((KP, D), jnp.float32),
            pltpu.VMEM((KP,), jnp.float32),
            pltpu.SemaphoreType.DMA,
        ],
    )
    def dot_kernel(h_hbm, src_hbm, dst_hbm, out_hbm, sa, sb, ra, rb, scv, sem):
        wid = lax.axis_index("s") * NC + lax.axis_index("c")
        wbase = wid * EPT

        def chunk_body(ci, carry):
            base = wbase + ci * KP
            pltpu.sync_copy(src_hbm.at[pl.ds(base, KP)], sa)
            pltpu.sync_copy(dst_hbm.at[pl.ds(base, KP)], sb)
            cp1 = pltpu.async_copy(h_hbm.at[sa], ra, sem)
            cp2 = pltpu.async_copy(h_hbm.at[sb], rb, sem)
            cp1.wait()
            cp2.wait()

            lane = lax.iota(jnp.int32, 16)

            def per_group(g, carry2):
                vec = jnp.zeros((16,), jnp.float32)
                for j in range(16):
                    k = g * 16 + j
                    accv = ra[k, pl.ds(0, 16)] * rb[k, pl.ds(0, 16)]
                    for c8 in range(1, D // 16):
                        sl = pl.ds(c8 * 16, 16)
                        accv = accv + ra[k, sl] * rb[k, sl]
                    vec = jnp.where(lane == j, jnp.sum(accv), vec)
                scv[pl.ds(g * 16, 16)] = vec
                return carry2

            lax.fori_loop(0, KP // 16, per_group, 0)
            pltpu.sync_copy(scv, out_hbm.at[pl.ds(base, KP)])
            return carry

        lax.fori_loop(0, NPCHUNKS, chunk_body, 0)

    return dot_kernel(h, src_all, dst_all)


def kernel(x, edge_index, neg_edge_index, Wp1, bp1, Ws1, Wn1, b1,
           Wp2, bp2, Ws2, Wn2, b2):
    src, dst = edge_index[0], edge_index[1]
    p1 = _dense1(x, Wp1, bp1, relu=True)
    n1 = _segmax_sc(p1, src, dst).reshape(N_NODES, D)
    h1 = _dense2(x, Ws1, n1, Wn1, b1, relu=True)
    p2 = _dense1(h1, Wp2, bp2, relu=True)
    n2 = _segmax_sc(p2, src, dst).reshape(N_NODES, D)
    h2 = _dense2(h1, Ws2, n2, Wn2, b2, relu=False)
    src_all = jnp.concatenate([src, neg_edge_index[0]])
    dst_all = jnp.concatenate([dst, neg_edge_index[1]])
    scores = _edge_dots_sc(h2, src_all, dst_all)
    pos = scores[:E].reshape(E, 1)
    neg = scores[E:].reshape(E, 1)
    return (pos, neg)


# speedup vs baseline: 2.4126x; 1.3482x over previous
import functools

import jax
import jax.numpy as jnp
from jax import lax
from jax.experimental import pallas as pl
from jax.experimental.pallas import tpu as pltpu
from jax.experimental.pallas import tpu_sc as plsc

N_NODES = 10000
E = 320000
D = 128
BN = 1000

NC = 2
NS = 16
NW = NC * NS
RPT = 313
LAST_ROWS = N_NODES - (NW - 1) * RPT
TRASH = RPT
CHUNK = 2560
NCHUNKS = E // CHUNK
NGROUPS = CHUNK // 16
K = 256
FB = 4096
OB = 2 * FB + 16
LCAP = E + OB - 16 + 2048
LCAP = ((LCAP + 2047) // 2048) * 2048


def _dense1(x, W, b, relu):
    def body(x_ref, w_ref, b_ref, o_ref):
        y = jnp.dot(x_ref[...], w_ref[...], preferred_element_type=jnp.float32)
        y = y + b_ref[...]
        if relu:
            y = jnp.maximum(y, 0.0)
        o_ref[...] = y

    n = x.shape[0]
    return pl.pallas_call(
        body,
        grid=(n // BN,),
        in_specs=[
            pl.BlockSpec((BN, D), lambda i: (i, 0)),
            pl.BlockSpec((D, D), lambda i: (0, 0)),
            pl.BlockSpec((1, D), lambda i: (0, 0)),
        ],
        out_specs=pl.BlockSpec((BN, D), lambda i: (i, 0)),
        out_shape=jax.ShapeDtypeStruct((n, D), jnp.float32),
    )(x, W, b.reshape(1, D))


def _dense2(x, Wa, n_agg, Wb, b, relu):
    def body(x_ref, wa_ref, n_ref, wb_ref, b_ref, o_ref):
        y = jnp.dot(x_ref[...], wa_ref[...], preferred_element_type=jnp.float32)
        y = y + jnp.dot(n_ref[...], wb_ref[...], preferred_element_type=jnp.float32)
        y = y + b_ref[...]
        if relu:
            y = jnp.maximum(y, 0.0)
        o_ref[...] = y

    n = x.shape[0]
    return pl.pallas_call(
        body,
        grid=(n // BN,),
        in_specs=[
            pl.BlockSpec((BN, D), lambda i: (i, 0)),
            pl.BlockSpec((D, D), lambda i: (0, 0)),
            pl.BlockSpec((BN, D), lambda i: (i, 0)),
            pl.BlockSpec((D, D), lambda i: (0, 0)),
            pl.BlockSpec((1, D), lambda i: (0, 0)),
        ],
        out_specs=pl.BlockSpec((BN, D), lambda i: (i, 0)),
        out_shape=jax.ShapeDtypeStruct((n, D), jnp.float32),
    )(x, Wa, n_agg, Wb, b.reshape(1, D))


def _partition_sc(src, dst):
    mesh = plsc.VectorSubcoreMesh(core_axis_name="c", subcore_axis_name="s")

    @functools.partial(
        pl.kernel,
        out_type=(
            jax.ShapeDtypeStruct((NW * LCAP,), jnp.int32),
            jax.ShapeDtypeStruct((NW * 16,), jnp.int32),
        ),
        mesh=mesh,
        compiler_params=pltpu.CompilerParams(needs_layout_passes=False),
        scratch_types=[
            pltpu.VMEM((CHUNK,), jnp.int32),
            pltpu.VMEM((CHUNK,), jnp.int32),
            pltpu.VMEM((OB,), jnp.int32),
            pltpu.VMEM((16,), jnp.int32),
        ],
    )
    def part_kernel(src_hbm, dst_hbm, opk_hbm, cnt_hbm, srcv, dstv, obuf, cntv):
        wid = lax.axis_index("s") * NC + lax.axis_index("c")
        lo = wid * RPT
        wbase = wid * LCAP

        def chunk_body(ci, carry):
            nacc, written = carry
            ebase = ci * CHUNK
            pltpu.sync_copy(src_hbm.at[pl.ds(ebase, CHUNK)], srcv)
            pltpu.sync_copy(dst_hbm.at[pl.ds(ebase, CHUNK)], dstv)

            def group_body(g, nacc):
                off = g * 16
                d16 = dstv[pl.ds(off, 16)]
                s16 = srcv[pl.ds(off, 16)]
                l16 = d16 - lo
                m = (l16 >= 0) & (l16 < RPT)
                packed = jnp.bitwise_or(jnp.left_shift(s16, 9), l16)
                plsc.store_compressed(obuf.at[pl.ds(nacc, 16)], packed, mask=m)
                return nacc + plsc.all_reduce_population_count(m)[0]

            nacc = lax.fori_loop(0, NGROUPS, group_body, nacc)

            full = nacc >= FB

            @pl.when(full)
            def _():
                off = pl.multiple_of(wbase + written, 8)
                pltpu.sync_copy(obuf.at[pl.ds(0, FB)],
                                opk_hbm.at[pl.ds(off, FB)])
                for g in range((OB - FB) // 16):
                    obuf[pl.ds(g * 16, 16)] = obuf[pl.ds(FB + g * 16, 16)]

            nacc = jnp.where(full, nacc - FB, nacc)
            written = jnp.where(full, written + FB, written)
            return (nacc, written)

        nacc, written = lax.fori_loop(
            0, NCHUNKS, chunk_body, (jnp.int32(0), jnp.int32(0)))

        lane = lax.iota(jnp.int32, 16)
        trash = jnp.full((16,), TRASH, jnp.int32)

        def pad_body(g, carry):
            off = g * 16
            keep = (lane + off) < nacc
            obuf[pl.ds(off, 16)] = jnp.where(keep, obuf[pl.ds(off, 16)], trash)
            return carry

        lax.fori_loop(0, OB // 16, pad_body, 0)

        @pl.when(nacc > 0)
        def _():
            off = pl.multiple_of(wbase + written, 8)
            pltpu.sync_copy(obuf.at[pl.ds(0, FB)],
                            opk_hbm.at[pl.ds(off, FB)])

        pcount = ((nacc + K - 1) // K) * K
        cntv[pl.ds(0, 16)] = jnp.zeros((16,), jnp.int32) + (written + pcount)
        pltpu.sync_copy(cntv, cnt_hbm.at[pl.ds(wid * 16, 16)])

    return part_kernel(src, dst)


def _segmax_sc(p, opk, cnt):
    mesh = plsc.VectorSubcoreMesh(core_axis_name="c", subcore_axis_name="s")

    @functools.partial(
        pl.kernel,
        out_type=jax.ShapeDtypeStruct((N_NODES * D,), jnp.float32),
        mesh=mesh,
        compiler_params=pltpu.CompilerParams(needs_layout_passes=False),
        scratch_types=[
            pltpu.VMEM((16,), jnp.int32),
            pltpu.VMEM((K + 16,), jnp.int32),
            pltpu.VMEM((K,), jnp.int32),
            pltpu.VMEM((K, D), jnp.float32),
            pltpu.VMEM(((RPT + 1) * D,), jnp.float32),
            pltpu.SemaphoreType.DMA,
        ],
    )
    def seg_kernel(p_hbm, opk_hbm, cnt_hbm, out_hbm,
                   cntv, pkv, gidx, rows, acc, sem):
        wid = lax.axis_index("s") * NC + lax.axis_index("c")
        lo = wid * RPT
        wbase = wid * LCAP

        def zero_body(i, carry):
            acc[pl.ds(i * 16, 16)] = jnp.zeros((16,), jnp.float32)
            return carry

        lax.fori_loop(0, (RPT + 1) * D // 16, zero_body, 0)

        pltpu.sync_copy(cnt_hbm.at[pl.ds(wid * 16, 16)], cntv)
        nb = cntv[pl.ds(0, 16)][0] // K

        def batch_body(b, carry):
            boff = pl.multiple_of(wbase + b * K, 8)
            pltpu.sync_copy(opk_hbm.at[pl.ds(boff, K)],
                            pkv.at[pl.ds(0, K)])
            for g in range(K // 16):
                gidx[pl.ds(g * 16, 16)] = lax.shift_right_logical(
                    pkv[pl.ds(g * 16, 16)], 9)
            pltpu.async_copy(p_hbm.at[gidx], rows, sem).wait()

            def per_edge(k, carry2):
                ld = pkv[pl.ds(k, 16)][0] & (512 - 1)
                base = ld * D
                for c8 in range(D // 16):
                    sl = pl.ds(base + c8 * 16, 16)
                    acc[sl] = jnp.maximum(acc[sl], rows[k, pl.ds(c8 * 16, 16)])
                return carry2

            lax.fori_loop(0, K, per_edge, 0)
            return carry

        lax.fori_loop(0, nb, batch_body, 0)

        @pl.when(wid < NW - 1)
        def _():
            pltpu.sync_copy(acc.at[pl.ds(0, RPT * D)],
                            out_hbm.at[pl.ds(lo * D, RPT * D)])

        @pl.when(wid == NW - 1)
        def _():
            pltpu.sync_copy(acc.at[pl.ds(0, LAST_ROWS * D)],
                            out_hbm.at[pl.ds(lo * D, LAST_ROWS * D)])

    return seg_kernel(p, opk, cnt)


EPT = 2 * E // NW
KP = 400
NPCHUNKS = EPT // KP


def _edge_dots_sc(h, src_all, dst_all):
    mesh = plsc.VectorSubcoreMesh(core_axis_name="c", subcore_axis_name="s")

    @functools.partial(
        pl.kernel,
        out_type=jax.ShapeDtypeStruct((2 * E,), jnp.float32),
        mesh=mesh,
        compiler_params=pltpu.CompilerParams(needs_layout_passes=False),
        scratch_types=[
            pltpu.VMEM((KP,), jnp.int32),
            pltpu.VMEM((KP,), jnp.int32),
            pltpu.VMEM((KP, D), jnp.float32),
            pltpu.VMEM((KP, D), jnp.float32),
            pltpu.VMEM((KP,), jnp.float32),
            pltpu.SemaphoreType.DMA,
        ],
    )
    def dot_kernel(h_hbm, src_hbm, dst_hbm, out_hbm, sa, sb, ra, rb, scv, sem):
        wid = lax.axis_index("s") * NC + lax.axis_index("c")
        wbase = wid * EPT

        def chunk_body(ci, carry):
            base = wbase + ci * KP
            pltpu.sync_copy(src_hbm.at[pl.ds(base, KP)], sa)
            pltpu.sync_copy(dst_hbm.at[pl.ds(base, KP)], sb)
            cp1 = pltpu.async_copy(h_hbm.at[sa], ra, sem)
            cp2 = pltpu.async_copy(h_hbm.at[sb], rb, sem)
            cp1.wait()
            cp2.wait()

            lane = lax.iota(jnp.int32, 16)

            def per_group(g, carry2):
                vec = jnp.zeros((16,), jnp.float32)
                for j in range(16):
                    k = g * 16 + j
                    accv = ra[k, pl.ds(0, 16)] * rb[k, pl.ds(0, 16)]
                    for c8 in range(1, D // 16):
                        sl = pl.ds(c8 * 16, 16)
                        accv = accv + ra[k, sl] * rb[k, sl]
                    vec = jnp.where(lane == j, jnp.sum(accv), vec)
                scv[pl.ds(g * 16, 16)] = vec
                return carry2

            lax.fori_loop(0, KP // 16, per_group, 0)
            pltpu.sync_copy(scv, out_hbm.at[pl.ds(base, KP)])
            return carry

        lax.fori_loop(0, NPCHUNKS, chunk_body, 0)

    return dot_kernel(h, src_all, dst_all)


def kernel(x, edge_index, neg_edge_index, Wp1, bp1, Ws1, Wn1, b1,
           Wp2, bp2, Ws2, Wn2, b2):
    src, dst = edge_index[0], edge_index[1]
    opk, cnt = _partition_sc(src, dst)
    p1 = _dense1(x, Wp1, bp1, relu=True)
    n1 = _segmax_sc(p1, opk, cnt).reshape(N_NODES, D)
    h1 = _dense2(x, Ws1, n1, Wn1, b1, relu=True)
    p2 = _dense1(h1, Wp2, bp2, relu=True)
    n2 = _segmax_sc(p2, opk, cnt).reshape(N_NODES, D)
    h2 = _dense2(h1, Ws2, n2, Wn2, b2, relu=False)
    src_all = jnp.concatenate([src, neg_edge_index[0]])
    dst_all = jnp.concatenate([dst, neg_edge_index[1]])
    scores = _edge_dots_sc(h2, src_all, dst_all)
    pos = scores[:E].reshape(E, 1)
    neg = scores[E:].reshape(E, 1)
    return (pos, neg)


# speedup vs baseline: 3.0583x; 1.2677x over previous
import functools

import jax
import jax.numpy as jnp
from jax import lax
from jax.experimental import pallas as pl
from jax.experimental.pallas import tpu as pltpu
from jax.experimental.pallas import tpu_sc as plsc

N_NODES = 10000
E = 320000
D = 128
BN = 1000

NC = 2
NS = 16
NW = NC * NS
RPT = 313
LAST_ROWS = N_NODES - (NW - 1) * RPT
TRASH = RPT
CHUNK = 2560
NCHUNKS = E // CHUNK
NGROUPS = CHUNK // 16
K = 256
FB = 4096
OB = 2 * FB + 16
LCAP = E + OB - 16 + 2048
LCAP = ((LCAP + 2047) // 2048) * 2048


def _dense1(x, W, b, relu):
    def body(x_ref, w_ref, b_ref, o_ref):
        y = jnp.dot(x_ref[...], w_ref[...], preferred_element_type=jnp.float32)
        y = y + b_ref[...]
        if relu:
            y = jnp.maximum(y, 0.0)
        o_ref[...] = y

    n = x.shape[0]
    return pl.pallas_call(
        body,
        grid=(n // BN,),
        in_specs=[
            pl.BlockSpec((BN, D), lambda i: (i, 0)),
            pl.BlockSpec((D, D), lambda i: (0, 0)),
            pl.BlockSpec((1, D), lambda i: (0, 0)),
        ],
        out_specs=pl.BlockSpec((BN, D), lambda i: (i, 0)),
        out_shape=jax.ShapeDtypeStruct((n, D), jnp.float32),
    )(x, W, b.reshape(1, D))


def _dense2(x, Wa, n_agg, Wb, b, relu):
    def body(x_ref, wa_ref, n_ref, wb_ref, b_ref, o_ref):
        y = jnp.dot(x_ref[...], wa_ref[...], preferred_element_type=jnp.float32)
        y = y + jnp.dot(n_ref[...], wb_ref[...], preferred_element_type=jnp.float32)
        y = y + b_ref[...]
        if relu:
            y = jnp.maximum(y, 0.0)
        o_ref[...] = y

    n = x.shape[0]
    return pl.pallas_call(
        body,
        grid=(n // BN,),
        in_specs=[
            pl.BlockSpec((BN, D), lambda i: (i, 0)),
            pl.BlockSpec((D, D), lambda i: (0, 0)),
            pl.BlockSpec((BN, D), lambda i: (i, 0)),
            pl.BlockSpec((D, D), lambda i: (0, 0)),
            pl.BlockSpec((1, D), lambda i: (0, 0)),
        ],
        out_specs=pl.BlockSpec((BN, D), lambda i: (i, 0)),
        out_shape=jax.ShapeDtypeStruct((n, D), jnp.float32),
    )(x, Wa, n_agg, Wb, b.reshape(1, D))


def _partition_sc(src, dst):
    mesh = plsc.VectorSubcoreMesh(core_axis_name="c", subcore_axis_name="s")

    @functools.partial(
        pl.kernel,
        out_type=(
            jax.ShapeDtypeStruct((NW * LCAP,), jnp.int32),
            jax.ShapeDtypeStruct((NW * 16,), jnp.int32),
        ),
        mesh=mesh,
        compiler_params=pltpu.CompilerParams(needs_layout_passes=False),
        scratch_types=[
            pltpu.VMEM((CHUNK,), jnp.int32),
            pltpu.VMEM((CHUNK,), jnp.int32),
            pltpu.VMEM((OB,), jnp.int32),
            pltpu.VMEM((16,), jnp.int32),
        ],
    )
    def part_kernel(src_hbm, dst_hbm, opk_hbm, cnt_hbm, srcv, dstv, obuf, cntv):
        wid = lax.axis_index("s") * NC + lax.axis_index("c")
        lo = wid * RPT
        wbase = wid * LCAP

        def chunk_body(ci, carry):
            nacc, written = carry
            ebase = ci * CHUNK
            pltpu.sync_copy(src_hbm.at[pl.ds(ebase, CHUNK)], srcv)
            pltpu.sync_copy(dst_hbm.at[pl.ds(ebase, CHUNK)], dstv)

            def group_body(g, nacc):
                off = g * 16
                d16 = dstv[pl.ds(off, 16)]
                s16 = srcv[pl.ds(off, 16)]
                l16 = d16 - lo
                m = (l16 >= 0) & (l16 < RPT)
                packed = jnp.bitwise_or(jnp.left_shift(s16, 9), l16)
                plsc.store_compressed(obuf.at[pl.ds(nacc, 16)], packed, mask=m)
                return nacc + plsc.all_reduce_population_count(m)[0]

            nacc = plsc.parallel_loop(0, NGROUPS, unroll=4,
                                      carry=nacc)(group_body)

            full = nacc >= FB

            @pl.when(full)
            def _():
                off = pl.multiple_of(wbase + written, 8)
                pltpu.sync_copy(obuf.at[pl.ds(0, FB)],
                                opk_hbm.at[pl.ds(off, FB)])
                for g in range((OB - FB) // 16):
                    obuf[pl.ds(g * 16, 16)] = obuf[pl.ds(FB + g * 16, 16)]

            nacc = jnp.where(full, nacc - FB, nacc)
            written = jnp.where(full, written + FB, written)
            return (nacc, written)

        nacc, written = lax.fori_loop(
            0, NCHUNKS, chunk_body, (jnp.int32(0), jnp.int32(0)))

        lane = lax.iota(jnp.int32, 16)
        trash = jnp.full((16,), TRASH, jnp.int32)

        def pad_body(g, carry):
            off = g * 16
            keep = (lane + off) < nacc
            obuf[pl.ds(off, 16)] = jnp.where(keep, obuf[pl.ds(off, 16)], trash)
            return carry

        lax.fori_loop(0, OB // 16, pad_body, 0)

        @pl.when(nacc > 0)
        def _():
            off = pl.multiple_of(wbase + written, 8)
            pltpu.sync_copy(obuf.at[pl.ds(0, FB)],
                            opk_hbm.at[pl.ds(off, FB)])

        pcount = ((nacc + K - 1) // K) * K
        cntv[pl.ds(0, 16)] = jnp.zeros((16,), jnp.int32) + (written + pcount)
        pltpu.sync_copy(cntv, cnt_hbm.at[pl.ds(wid * 16, 16)])

    return part_kernel(src, dst)


def _segmax_sc(p, opk, cnt):
    mesh = plsc.VectorSubcoreMesh(core_axis_name="c", subcore_axis_name="s")

    @functools.partial(
        pl.kernel,
        out_type=jax.ShapeDtypeStruct((N_NODES * D,), jnp.float32),
        mesh=mesh,
        compiler_params=pltpu.CompilerParams(needs_layout_passes=False),
        scratch_types=[
            pltpu.VMEM((16,), jnp.int32),
            pltpu.VMEM((K + 16,), jnp.int32),
            pltpu.VMEM((K + 16,), jnp.int32),
            pltpu.VMEM((K,), jnp.int32),
            pltpu.VMEM((K,), jnp.int32),
            pltpu.VMEM((K, D), jnp.float32),
            pltpu.VMEM((K, D), jnp.float32),
            pltpu.VMEM(((RPT + 1) * D,), jnp.float32),
            pltpu.SemaphoreType.DMA,
            pltpu.SemaphoreType.DMA,
        ],
    )
    def seg_kernel(p_hbm, opk_hbm, cnt_hbm, out_hbm,
                   cntv, pkv0, pkv1, gidx0, gidx1, rows0, rows1, acc,
                   sem0, sem1):
        wid = lax.axis_index("s") * NC + lax.axis_index("c")
        lo = wid * RPT
        wbase = wid * LCAP

        def zero_body(i):
            acc[pl.ds(i * 16, 16)] = jnp.zeros((16,), jnp.float32)

        plsc.parallel_loop(0, (RPT + 1) * D // 16, unroll=4)(zero_body)

        pltpu.sync_copy(cnt_hbm.at[pl.ds(wid * 16, 16)], cntv)
        nb = cntv[pl.ds(0, 16)][0] // K

        bufs = ((pkv0, gidx0, rows0, sem0), (pkv1, gidx1, rows1, sem1))

        def start(b, pkv, gidx, rows, sem):
            boff = pl.multiple_of(wbase + b * K, 8)
            pltpu.sync_copy(opk_hbm.at[pl.ds(boff, K)], pkv.at[pl.ds(0, K)])
            for g in range(K // 16):
                gidx[pl.ds(g * 16, 16)] = lax.shift_right_logical(
                    pkv[pl.ds(g * 16, 16)], 9)
            pltpu.make_async_copy(p_hbm.at[gidx], rows, sem).start()

        def rmw(pkv, gidx, rows, sem):
            pltpu.make_async_copy(p_hbm.at[gidx], rows, sem).wait()

            def group_rmw(g, carry):
                ldv = jnp.left_shift(pkv[pl.ds(g * 16, 16)] & (512 - 1), 7)
                for j in range(16):
                    base = ldv[j]
                    k = g * 16 + j
                    for c8 in range(D // 16):
                        sl = pl.ds(base + c8 * 16, 16)
                        acc[sl] = jnp.maximum(acc[sl],
                                              rows[k, pl.ds(c8 * 16, 16)])
                return carry

            lax.fori_loop(0, K // 16, group_rmw, 0)

        @pl.when(nb > 0)
        def _():
            start(0, *bufs[0])

        def pair_body(i, carry):
            b0 = 2 * i

            @pl.when(b0 + 1 < nb)
            def _():
                start(b0 + 1, *bufs[1])

            rmw(*bufs[0])

            @pl.when(b0 + 2 < nb)
            def _():
                start(b0 + 2, *bufs[0])

            @pl.when(b0 + 1 < nb)
            def _():
                rmw(*bufs[1])

            return carry

        lax.fori_loop(0, (nb + 1) // 2, pair_body, 0)

        @pl.when(wid < NW - 1)
        def _():
            pltpu.sync_copy(acc.at[pl.ds(0, RPT * D)],
                            out_hbm.at[pl.ds(lo * D, RPT * D)])

        @pl.when(wid == NW - 1)
        def _():
            pltpu.sync_copy(acc.at[pl.ds(0, LAST_ROWS * D)],
                            out_hbm.at[pl.ds(lo * D, LAST_ROWS * D)])

    return seg_kernel(p, opk, cnt)


EPT = 2 * E // NW
KP = 80
NPCHUNKS = EPT // KP


def _edge_dots_sc(h, src_all, dst_all):
    mesh = plsc.VectorSubcoreMesh(core_axis_name="c", subcore_axis_name="s")

    @functools.partial(
        pl.kernel,
        out_type=jax.ShapeDtypeStruct((2 * E,), jnp.float32),
        mesh=mesh,
        compiler_params=pltpu.CompilerParams(needs_layout_passes=False),
        scratch_types=[
            pltpu.VMEM((KP,), jnp.int32),
            pltpu.VMEM((KP,), jnp.int32),
            pltpu.VMEM((KP,), jnp.int32),
            pltpu.VMEM((KP,), jnp.int32),
            pltpu.VMEM((KP, D), jnp.float32),
            pltpu.VMEM((KP, D), jnp.float32),
            pltpu.VMEM((KP, D), jnp.float32),
            pltpu.VMEM((KP, D), jnp.float32),
            pltpu.VMEM((KP,), jnp.float32),
            pltpu.SemaphoreType.DMA,
            pltpu.SemaphoreType.DMA,
        ],
    )
    def dot_kernel(h_hbm, src_hbm, dst_hbm, out_hbm,
                   sa0, sb0, sa1, sb1, ra0, rb0, ra1, rb1, scv, sem0, sem1):
        wid = lax.axis_index("s") * NC + lax.axis_index("c")
        wbase = wid * EPT
        lane = lax.iota(jnp.int32, 16)

        def start(c, sa, sb, ra, rb, sem):
            base = pl.multiple_of(wbase + c * KP, 8)
            pltpu.sync_copy(src_hbm.at[pl.ds(base, KP)], sa)
            pltpu.sync_copy(dst_hbm.at[pl.ds(base, KP)], sb)
            pltpu.make_async_copy(h_hbm.at[sa], ra, sem).start()
            pltpu.make_async_copy(h_hbm.at[sb], rb, sem).start()

        def compute(c, sa, sb, ra, rb, sem):
            pltpu.make_async_copy(h_hbm.at[sa], ra, sem).wait()
            pltpu.make_async_copy(h_hbm.at[sb], rb, sem).wait()

            def per_group(g):
                vec = jnp.zeros((16,), jnp.float32)
                for j in range(16):
                    k = g * 16 + j
                    accv = ra[k, pl.ds(0, 16)] * rb[k, pl.ds(0, 16)]
                    for c8 in range(1, D // 16):
                        sl = pl.ds(c8 * 16, 16)
                        accv = accv + ra[k, sl] * rb[k, sl]
                    vec = jnp.where(lane == j, jnp.sum(accv), vec)
                scv[pl.ds(g * 16, 16)] = vec

            plsc.parallel_loop(0, KP // 16)(per_group)
            base = pl.multiple_of(wbase + c * KP, 8)
            pltpu.sync_copy(scv, out_hbm.at[pl.ds(base, KP)])

        bufs = ((sa0, sb0, ra0, rb0, sem0), (sa1, sb1, ra1, rb1, sem1))

        start(0, *bufs[0])

        def pair_body(i, carry):
            c0 = 2 * i
            start(c0 + 1, *bufs[1])
            compute(c0, *bufs[0])

            @pl.when(c0 + 2 < NPCHUNKS)
            def _():
                start(c0 + 2, *bufs[0])

            compute(c0 + 1, *bufs[1])
            return carry

        lax.fori_loop(0, NPCHUNKS // 2, pair_body, 0)

    return dot_kernel(h, src_all, dst_all)


def kernel(x, edge_index, neg_edge_index, Wp1, bp1, Ws1, Wn1, b1,
           Wp2, bp2, Ws2, Wn2, b2):
    src, dst = edge_index[0], edge_index[1]
    opk, cnt = _partition_sc(src, dst)
    p1 = _dense1(x, Wp1, bp1, relu=True)
    n1 = _segmax_sc(p1, opk, cnt).reshape(N_NODES, D)
    h1 = _dense2(x, Ws1, n1, Wn1, b1, relu=True)
    p2 = _dense1(h1, Wp2, bp2, relu=True)
    n2 = _segmax_sc(p2, opk, cnt).reshape(N_NODES, D)
    h2 = _dense2(h1, Ws2, n2, Wn2, b2, relu=False)
    src_all = jnp.concatenate([src, neg_edge_index[0]])
    dst_all = jnp.concatenate([dst, neg_edge_index[1]])
    scores = _edge_dots_sc(h2, src_all, dst_all)
    pos = scores[:E].reshape(E, 1)
    neg = scores[E:].reshape(E, 1)
    return (pos, neg)


# speedup vs baseline: 4.6312x; 1.5143x over previous
import functools

import jax
import jax.numpy as jnp
from jax import lax
from jax.experimental import pallas as pl
from jax.experimental.pallas import tpu as pltpu
from jax.experimental.pallas import tpu_sc as plsc

N_NODES = 10000
E = 320000
D = 128
DW = D // 2
BN = 1000

NC = 2
NS = 16
NW = NC * NS
RPT = 313
LAST_ROWS = N_NODES - (NW - 1) * RPT
TRASH = RPT
CHUNK = 2560
NCHUNKS = E // CHUNK
NGROUPS = CHUNK // 16
K = 256
FB = 4096
OB = 2 * FB + 16
LCAP = E + OB - 16 + 2048
LCAP = ((LCAP + 2047) // 2048) * 2048


def _dense_pool(x, W, b):
    def body(x_ref, w_ref, b_ref, o_ref):
        y = jnp.dot(x_ref[...], w_ref[...], preferred_element_type=jnp.float32)
        y = jnp.maximum(y + b_ref[...], 0.0)
        o_ref[...] = y.astype(jnp.bfloat16)

    n = x.shape[0]
    return pl.pallas_call(
        body,
        grid=(n // BN,),
        in_specs=[
            pl.BlockSpec((BN, D), lambda i: (i, 0)),
            pl.BlockSpec((D, D), lambda i: (0, 0)),
            pl.BlockSpec((1, D), lambda i: (0, 0)),
        ],
        out_specs=pl.BlockSpec((BN, D), lambda i: (i, 0)),
        out_shape=jax.ShapeDtypeStruct((n, D), jnp.bfloat16),
    )(x, W, b.reshape(1, D))


def _dense2(x, Wa, n_agg, Wb, b, relu, out_bf16):
    def body(x_ref, wa_ref, n_ref, wb_ref, b_ref, o_ref):
        y = jnp.dot(x_ref[...], wa_ref[...], preferred_element_type=jnp.float32)
        nv = n_ref[...].astype(jnp.float32)
        y = y + jnp.dot(nv, wb_ref[...], preferred_element_type=jnp.float32)
        y = y + b_ref[...]
        if relu:
            y = jnp.maximum(y, 0.0)
        o_ref[...] = y.astype(o_ref.dtype)

    n = x.shape[0]
    odtype = jnp.bfloat16 if out_bf16 else jnp.float32
    return pl.pallas_call(
        body,
        grid=(n // BN,),
        in_specs=[
            pl.BlockSpec((BN, D), lambda i: (i, 0)),
            pl.BlockSpec((D, D), lambda i: (0, 0)),
            pl.BlockSpec((BN, D), lambda i: (i, 0)),
            pl.BlockSpec((D, D), lambda i: (0, 0)),
            pl.BlockSpec((1, D), lambda i: (0, 0)),
        ],
        out_specs=pl.BlockSpec((BN, D), lambda i: (i, 0)),
        out_shape=jax.ShapeDtypeStruct((n, D), odtype),
    )(x, Wa, n_agg, Wb, b.reshape(1, D))


def _partition_sc(src, dst):
    mesh = plsc.VectorSubcoreMesh(core_axis_name="c", subcore_axis_name="s")

    @functools.partial(
        pl.kernel,
        out_type=(
            jax.ShapeDtypeStruct((NW * LCAP,), jnp.int32),
            jax.ShapeDtypeStruct((NW * 16,), jnp.int32),
        ),
        mesh=mesh,
        compiler_params=pltpu.CompilerParams(needs_layout_passes=False),
        scratch_types=[
            pltpu.VMEM((CHUNK,), jnp.int32),
            pltpu.VMEM((CHUNK,), jnp.int32),
            pltpu.VMEM((OB,), jnp.int32),
            pltpu.VMEM((16,), jnp.int32),
        ],
    )
    def part_kernel(src_hbm, dst_hbm, opk_hbm, cnt_hbm, srcv, dstv, obuf, cntv):
        wid = lax.axis_index("s") * NC + lax.axis_index("c")
        lo = wid * RPT
        wbase = wid * LCAP

        def chunk_body(ci, carry):
            nacc, written = carry
            ebase = ci * CHUNK
            pltpu.sync_copy(src_hbm.at[pl.ds(ebase, CHUNK)], srcv)
            pltpu.sync_copy(dst_hbm.at[pl.ds(ebase, CHUNK)], dstv)

            def group_body(g, nacc):
                off = g * 16
                d16 = dstv[pl.ds(off, 16)]
                s16 = srcv[pl.ds(off, 16)]
                l16 = d16 - lo
                m = (l16 >= 0) & (l16 < RPT)
                packed = jnp.bitwise_or(jnp.left_shift(s16, 9), l16)
                plsc.store_compressed(obuf.at[pl.ds(nacc, 16)], packed, mask=m)
                return nacc + plsc.all_reduce_population_count(m)[0]

            nacc = plsc.parallel_loop(0, NGROUPS, unroll=4,
                                      carry=nacc)(group_body)

            full = nacc >= FB

            @pl.when(full)
            def _():
                off = pl.multiple_of(wbase + written, 8)
                pltpu.sync_copy(obuf.at[pl.ds(0, FB)],
                                opk_hbm.at[pl.ds(off, FB)])
                for g in range((OB - FB) // 16):
                    obuf[pl.ds(g * 16, 16)] = obuf[pl.ds(FB + g * 16, 16)]

            nacc = jnp.where(full, nacc - FB, nacc)
            written = jnp.where(full, written + FB, written)
            return (nacc, written)

        nacc, written = lax.fori_loop(
            0, NCHUNKS, chunk_body, (jnp.int32(0), jnp.int32(0)))

        lane = lax.iota(jnp.int32, 16)
        trash = jnp.full((16,), TRASH, jnp.int32)

        def pad_body(g, carry):
            off = g * 16
            keep = (lane + off) < nacc
            obuf[pl.ds(off, 16)] = jnp.where(keep, obuf[pl.ds(off, 16)], trash)
            return carry

        lax.fori_loop(0, OB // 16, pad_body, 0)

        @pl.when(nacc > 0)
        def _():
            off = pl.multiple_of(wbase + written, 8)
            pltpu.sync_copy(obuf.at[pl.ds(0, FB)],
                            opk_hbm.at[pl.ds(off, FB)])

        pcount = ((nacc + K - 1) // K) * K
        cntv[pl.ds(0, 16)] = jnp.zeros((16,), jnp.int32) + (written + pcount)
        pltpu.sync_copy(cntv, cnt_hbm.at[pl.ds(wid * 16, 16)])

    return part_kernel(src, dst)


def _segmax_sc(p, opk, cnt):
    mesh = plsc.VectorSubcoreMesh(core_axis_name="c", subcore_axis_name="s")

    @functools.partial(
        pl.kernel,
        out_type=jax.ShapeDtypeStruct((N_NODES * DW,), jnp.int32),
        mesh=mesh,
        compiler_params=pltpu.CompilerParams(needs_layout_passes=False,
                                             use_tc_tiling_on_sc=False),
        scratch_types=[
            pltpu.VMEM((16,), jnp.int32),
            pltpu.VMEM((K + 16,), jnp.int32),
            pltpu.VMEM((K + 16,), jnp.int32),
            pltpu.VMEM((K,), jnp.int32),
            pltpu.VMEM((K,), jnp.int32),
            pltpu.VMEM((K, DW), jnp.int32),
            pltpu.VMEM((K, DW), jnp.int32),
            pltpu.VMEM(((RPT + 1) * DW,), jnp.int32),
            pltpu.VMEM(((RPT + 1) * DW,), jnp.int32),
            pltpu.VMEM(((RPT + 1) * DW,), jnp.int32),
            pltpu.SemaphoreType.DMA,
            pltpu.SemaphoreType.DMA,
        ],
    )
    def seg_kernel(p_hbm, opk_hbm, cnt_hbm, out_hbm,
                   cntv, pkv0, pkv1, gidx0, gidx1, rows0, rows1,
                   acc0, acc1, acc2, sem0, sem1):
        wid = lax.axis_index("s") * NC + lax.axis_index("c")
        lo = wid * RPT
        wbase = wid * LCAP
        accs = (acc0, acc1, acc2)

        def zero_body(i):
            z = jnp.zeros((16,), jnp.int32)
            for a in accs:
                a[pl.ds(i * 16, 16)] = z

        plsc.parallel_loop(0, (RPT + 1) * DW // 16, unroll=2)(zero_body)

        pltpu.sync_copy(cnt_hbm.at[pl.ds(wid * 16, 16)], cntv)
        nb = cntv[pl.ds(0, 16)][0] // K

        bufs = ((pkv0, gidx0, rows0, sem0), (pkv1, gidx1, rows1, sem1))

        def start(b, pkv, gidx, rows, sem):
            boff = pl.multiple_of(wbase + b * K, 8)
            pltpu.sync_copy(opk_hbm.at[pl.ds(boff, K)], pkv.at[pl.ds(0, K)])
            for g in range(K // 16):
                gidx[pl.ds(g * 16, 16)] = lax.shift_right_logical(
                    pkv[pl.ds(g * 16, 16)], 9)
            pltpu.make_async_copy(p_hbm.at[gidx], rows, sem).start()

        def rmw(pkv, gidx, rows, sem):
            pltpu.make_async_copy(p_hbm.at[gidx], rows, sem).wait()

            def group_rmw(g, carry):
                ldv = (pkv[pl.ds(g * 16, 16)] & (512 - 1)) * DW
                for j in range(16):
                    base = ldv[j]
                    k = g * 16 + j
                    a = accs[j % 3]
                    for c4 in range(DW // 16):
                        sl = pl.ds(base + c4 * 16, 16)
                        va = plsc.bitcast(rows[k, pl.ds(c4 * 16, 16)],
                                          jnp.bfloat16)
                        vo = plsc.bitcast(a[sl], jnp.bfloat16)
                        a[sl] = plsc.bitcast(jnp.maximum(vo, va), jnp.int32)
                return carry

            lax.fori_loop(0, K // 16, group_rmw, 0)

        @pl.when(nb > 0)
        def _():
            start(0, *bufs[0])

        def pair_body(i, carry):
            b0 = 2 * i

            @pl.when(b0 + 1 < nb)
            def _():
                start(b0 + 1, *bufs[1])

            rmw(*bufs[0])

            @pl.when(b0 + 2 < nb)
            def _():
                start(b0 + 2, *bufs[0])

            @pl.when(b0 + 1 < nb)
            def _():
                rmw(*bufs[1])

            return carry

        lax.fori_loop(0, (nb + 1) // 2, pair_body, 0)

        def merge_body(i):
            sl = pl.ds(i * 16, 16)
            m0 = jnp.maximum(plsc.bitcast(acc0[sl], jnp.bfloat16),
                             plsc.bitcast(acc1[sl], jnp.bfloat16))
            m1 = plsc.bitcast(acc2[sl], jnp.bfloat16)
            acc0[sl] = plsc.bitcast(jnp.maximum(m0, m1), jnp.int32)

        plsc.parallel_loop(0, RPT * DW // 16, unroll=2)(merge_body)

        @pl.when(wid < NW - 1)
        def _():
            pltpu.sync_copy(acc0.at[pl.ds(0, RPT * DW)],
                            out_hbm.at[pl.ds(lo * DW, RPT * DW)])

        @pl.when(wid == NW - 1)
        def _():
            pltpu.sync_copy(acc0.at[pl.ds(0, LAST_ROWS * DW)],
                            out_hbm.at[pl.ds(lo * DW, LAST_ROWS * DW)])

    return seg_kernel(p, opk, cnt)


EPT = 2 * E // NW
KP = 160
NPCHUNKS = EPT // KP


def _edge_dots_sc(h, src_all, dst_all):
    mesh = plsc.VectorSubcoreMesh(core_axis_name="c", subcore_axis_name="s")

    @functools.partial(
        pl.kernel,
        out_type=jax.ShapeDtypeStruct((2 * E,), jnp.float32),
        mesh=mesh,
        compiler_params=pltpu.CompilerParams(needs_layout_passes=False,
                                             use_tc_tiling_on_sc=False),
        scratch_types=[
            pltpu.VMEM((KP,), jnp.int32),
            pltpu.VMEM((KP,), jnp.int32),
            pltpu.VMEM((KP,), jnp.int32),
            pltpu.VMEM((KP,), jnp.int32),
            pltpu.VMEM((KP, DW), jnp.int32),
            pltpu.VMEM((KP, DW), jnp.int32),
            pltpu.VMEM((KP, DW), jnp.int32),
            pltpu.VMEM((KP, DW), jnp.int32),
            pltpu.VMEM((KP,), jnp.float32),
            pltpu.SemaphoreType.DMA,
            pltpu.SemaphoreType.DMA,
        ],
    )
    def dot_kernel(h_hbm, src_hbm, dst_hbm, out_hbm,
                   sa0, sb0, sa1, sb1, ra0, rb0, ra1, rb1, scv, sem0, sem1):
        wid = lax.axis_index("s") * NC + lax.axis_index("c")
        wbase = wid * EPT
        lane = lax.iota(jnp.int32, 16)

        def start(c, sa, sb, ra, rb, sem):
            base = pl.multiple_of(wbase + c * KP, 8)
            pltpu.sync_copy(src_hbm.at[pl.ds(base, KP)], sa)
            pltpu.sync_copy(dst_hbm.at[pl.ds(base, KP)], sb)
            pltpu.make_async_copy(h_hbm.at[sa], ra, sem).start()
            pltpu.make_async_copy(h_hbm.at[sb], rb, sem).start()

        def compute(c, sa, sb, ra, rb, sem):
            pltpu.make_async_copy(h_hbm.at[sa], ra, sem).wait()
            pltpu.make_async_copy(h_hbm.at[sb], rb, sem).wait()

            def per_group(g):
                vec = jnp.zeros((16,), jnp.float32)
                for j in range(16):
                    k = g * 16 + j
                    accv = jnp.zeros((16,), jnp.float32)
                    for c4 in range(DW // 16):
                        sl = pl.ds(c4 * 16, 16)
                        va = plsc.bitcast(ra[k, sl], jnp.bfloat16)
                        vb = plsc.bitcast(rb[k, sl], jnp.bfloat16)
                        prod = va * vb
                        p0, p1 = plsc.unpack(
                            prod, format=plsc.PackFormat.INTERLEAVED)
                        accv = accv + p0 + p1
                    vec = jnp.where(lane == j, jnp.sum(accv), vec)
                scv[pl.ds(g * 16, 16)] = vec

            plsc.parallel_loop(0, KP // 16)(per_group)
            base = pl.multiple_of(wbase + c * KP, 8)
            pltpu.sync_copy(scv, out_hbm.at[pl.ds(base, KP)])

        bufs = ((sa0, sb0, ra0, rb0, sem0), (sa1, sb1, ra1, rb1, sem1))

        start(0, *bufs[0])

        def pair_body(i, carry):
            c0 = 2 * i

            @pl.when(c0 + 1 < NPCHUNKS)
            def _():
                start(c0 + 1, *bufs[1])

            compute(c0, *bufs[0])

            @pl.when(c0 + 2 < NPCHUNKS)
            def _():
                start(c0 + 2, *bufs[0])

            @pl.when(c0 + 1 < NPCHUNKS)
            def _():
                compute(c0 + 1, *bufs[1])

            return carry

        lax.fori_loop(0, (NPCHUNKS + 1) // 2, pair_body, 0)

    return dot_kernel(h, src_all, dst_all)


def _b16_as_i32(t):
    return lax.bitcast_convert_type(t.reshape(N_NODES, DW, 2), jnp.int32)


def _i32_as_b16(t):
    return lax.bitcast_convert_type(
        t.reshape(N_NODES, DW), jnp.bfloat16).reshape(N_NODES, D)


def kernel(x, edge_index, neg_edge_index, Wp1, bp1, Ws1, Wn1, b1,
           Wp2, bp2, Ws2, Wn2, b2):
    src, dst = edge_index[0], edge_index[1]
    opk, cnt = _partition_sc(src, dst)
    p1 = _b16_as_i32(_dense_pool(x, Wp1, bp1))
    n1 = _i32_as_b16(_segmax_sc(p1, opk, cnt))
    h1 = _dense2(x, Ws1, n1, Wn1, b1, relu=True, out_bf16=False)
    p2 = _b16_as_i32(_dense_pool(h1, Wp2, bp2))
    n2 = _i32_as_b16(_segmax_sc(p2, opk, cnt))
    h2 = _b16_as_i32(_dense2(h1, Ws2, n2, Wn2, b2, relu=False, out_bf16=True))
    src_all = jnp.concatenate([src, neg_edge_index[0]])
    dst_all = jnp.concatenate([dst, neg_edge_index[1]])
    scores = _edge_dots_sc(h2, src_all, dst_all)
    pos = scores[:E].reshape(E, 1)
    neg = scores[E:].reshape(E, 1)
    return (pos, neg)


# speedup vs baseline: 4.6367x; 1.0012x over previous
import functools

import jax
import jax.numpy as jnp
from jax import lax
from jax.experimental import pallas as pl
from jax.experimental.pallas import tpu as pltpu
from jax.experimental.pallas import tpu_sc as plsc

N_NODES = 10000
E = 320000
D = 128
DW = D // 2
BN = 1000

NC = 2
NS = 16
NW = NC * NS
RPT = 313
LAST_ROWS = N_NODES - (NW - 1) * RPT
TRASH = RPT
CHUNK = 2560
NCHUNKS = E // CHUNK
NGROUPS = CHUNK // 16
K = 256
FB = 4096
OB = 2 * FB + 16
LCAP = E + OB - 16 + 2048
LCAP = ((LCAP + 2047) // 2048) * 2048


def _dense_pool(x, W, b):
    def body(x_ref, w_ref, b_ref, o_ref):
        y = jnp.dot(x_ref[...], w_ref[...], preferred_element_type=jnp.float32)
        y = jnp.maximum(y + b_ref[...], 0.0)
        o_ref[...] = y.astype(jnp.bfloat16)

    n = x.shape[0]
    return pl.pallas_call(
        body,
        grid=(n // BN,),
        in_specs=[
            pl.BlockSpec((BN, D), lambda i: (i, 0)),
            pl.BlockSpec((D, D), lambda i: (0, 0)),
            pl.BlockSpec((1, D), lambda i: (0, 0)),
        ],
        out_specs=pl.BlockSpec((BN, D), lambda i: (i, 0)),
        out_shape=jax.ShapeDtypeStruct((n, D), jnp.bfloat16),
    )(x, W, b.reshape(1, D))


def _dense2(x, Wa, n_agg, Wb, b, relu, out_bf16):
    def body(x_ref, wa_ref, n_ref, wb_ref, b_ref, o_ref):
        y = jnp.dot(x_ref[...], wa_ref[...], preferred_element_type=jnp.float32)
        nv = n_ref[...].astype(jnp.float32)
        y = y + jnp.dot(nv, wb_ref[...], preferred_element_type=jnp.float32)
        y = y + b_ref[...]
        if relu:
            y = jnp.maximum(y, 0.0)
        o_ref[...] = y.astype(o_ref.dtype)

    n = x.shape[0]
    odtype = jnp.bfloat16 if out_bf16 else jnp.float32
    return pl.pallas_call(
        body,
        grid=(n // BN,),
        in_specs=[
            pl.BlockSpec((BN, D), lambda i: (i, 0)),
            pl.BlockSpec((D, D), lambda i: (0, 0)),
            pl.BlockSpec((BN, D), lambda i: (i, 0)),
            pl.BlockSpec((D, D), lambda i: (0, 0)),
            pl.BlockSpec((1, D), lambda i: (0, 0)),
        ],
        out_specs=pl.BlockSpec((BN, D), lambda i: (i, 0)),
        out_shape=jax.ShapeDtypeStruct((n, D), odtype),
    )(x, Wa, n_agg, Wb, b.reshape(1, D))


def _partition_sc(src, dst):
    mesh = plsc.VectorSubcoreMesh(core_axis_name="c", subcore_axis_name="s")

    @functools.partial(
        pl.kernel,
        out_type=(
            jax.ShapeDtypeStruct((NW * LCAP,), jnp.int32),
            jax.ShapeDtypeStruct((NW * 16,), jnp.int32),
        ),
        mesh=mesh,
        compiler_params=pltpu.CompilerParams(needs_layout_passes=False),
        scratch_types=[
            pltpu.VMEM((CHUNK,), jnp.int32),
            pltpu.VMEM((CHUNK,), jnp.int32),
            pltpu.VMEM((OB,), jnp.int32),
            pltpu.VMEM((16,), jnp.int32),
        ],
    )
    def part_kernel(src_hbm, dst_hbm, opk_hbm, cnt_hbm, srcv, dstv, obuf, cntv):
        wid = lax.axis_index("s") * NC + lax.axis_index("c")
        lo = wid * RPT
        wbase = wid * LCAP

        def chunk_body(ci, carry):
            nacc, written = carry
            ebase = ci * CHUNK
            pltpu.sync_copy(src_hbm.at[pl.ds(ebase, CHUNK)], srcv)
            pltpu.sync_copy(dst_hbm.at[pl.ds(ebase, CHUNK)], dstv)

            def group_body(g, nacc):
                off = g * 16
                d16 = dstv[pl.ds(off, 16)]
                s16 = srcv[pl.ds(off, 16)]
                l16 = d16 - lo
                m = (l16 >= 0) & (l16 < RPT)
                packed = jnp.bitwise_or(jnp.left_shift(s16, 9), l16)
                plsc.store_compressed(obuf.at[pl.ds(nacc, 16)], packed, mask=m)
                return nacc + plsc.all_reduce_population_count(m)[0]

            nacc = plsc.parallel_loop(0, NGROUPS, unroll=4,
                                      carry=nacc)(group_body)

            full = nacc >= FB

            @pl.when(full)
            def _():
                off = pl.multiple_of(wbase + written, 8)
                pltpu.sync_copy(obuf.at[pl.ds(0, FB)],
                                opk_hbm.at[pl.ds(off, FB)])
                for g in range((OB - FB) // 16):
                    obuf[pl.ds(g * 16, 16)] = obuf[pl.ds(FB + g * 16, 16)]

            nacc = jnp.where(full, nacc - FB, nacc)
            written = jnp.where(full, written + FB, written)
            return (nacc, written)

        nacc, written = lax.fori_loop(
            0, NCHUNKS, chunk_body, (jnp.int32(0), jnp.int32(0)))

        lane = lax.iota(jnp.int32, 16)
        trash = jnp.full((16,), TRASH, jnp.int32)

        def pad_body(g, carry):
            off = g * 16
            keep = (lane + off) < nacc
            obuf[pl.ds(off, 16)] = jnp.where(keep, obuf[pl.ds(off, 16)], trash)
            return carry

        lax.fori_loop(0, OB // 16, pad_body, 0)

        @pl.when(nacc > 0)
        def _():
            off = pl.multiple_of(wbase + written, 8)
            pltpu.sync_copy(obuf.at[pl.ds(0, FB)],
                            opk_hbm.at[pl.ds(off, FB)])

        pcount = ((nacc + K - 1) // K) * K
        cntv[pl.ds(0, 16)] = jnp.zeros((16,), jnp.int32) + (written + pcount)
        pltpu.sync_copy(cntv, cnt_hbm.at[pl.ds(wid * 16, 16)])

    return part_kernel(src, dst)


def _segmax_sc(p, opk, cnt):
    mesh = plsc.VectorSubcoreMesh(core_axis_name="c", subcore_axis_name="s")

    @functools.partial(
        pl.kernel,
        out_type=jax.ShapeDtypeStruct((N_NODES * DW,), jnp.int32),
        mesh=mesh,
        compiler_params=pltpu.CompilerParams(needs_layout_passes=False,
                                             use_tc_tiling_on_sc=False),
        scratch_types=[
            pltpu.VMEM((16,), jnp.int32),
            pltpu.VMEM((K + 16,), jnp.int32),
            pltpu.VMEM((K + 16,), jnp.int32),
            pltpu.VMEM((K,), jnp.int32),
            pltpu.VMEM((K,), jnp.int32),
            pltpu.VMEM((K, DW), jnp.int32),
            pltpu.VMEM((K, DW), jnp.int32),
            pltpu.VMEM(((RPT + 1) * DW,), jnp.int32),
            pltpu.VMEM(((RPT + 1) * DW,), jnp.int32),
            pltpu.VMEM(((RPT + 1) * DW,), jnp.int32),
            pltpu.SemaphoreType.DMA,
            pltpu.SemaphoreType.DMA,
        ],
    )
    def seg_kernel(p_hbm, opk_hbm, cnt_hbm, out_hbm,
                   cntv, pkv0, pkv1, gidx0, gidx1, rows0, rows1,
                   acc0, acc1, acc2, sem0, sem1):
        wid = lax.axis_index("s") * NC + lax.axis_index("c")
        lo = wid * RPT
        wbase = wid * LCAP
        accs = (acc0, acc1, acc2)

        def zero_body(i):
            z = jnp.zeros((16,), jnp.int32)
            for a in accs:
                a[pl.ds(i * 16, 16)] = z

        plsc.parallel_loop(0, (RPT + 1) * DW // 16, unroll=2)(zero_body)

        pltpu.sync_copy(cnt_hbm.at[pl.ds(wid * 16, 16)], cntv)
        nb = cntv[pl.ds(0, 16)][0] // K

        bufs = ((pkv0, gidx0, rows0, sem0), (pkv1, gidx1, rows1, sem1))

        def start(b, pkv, gidx, rows, sem):
            boff = pl.multiple_of(wbase + b * K, 8)
            pltpu.sync_copy(opk_hbm.at[pl.ds(boff, K)], pkv.at[pl.ds(0, K)])
            for g in range(K // 16):
                gidx[pl.ds(g * 16, 16)] = lax.shift_right_logical(
                    pkv[pl.ds(g * 16, 16)], 9)
            pltpu.make_async_copy(p_hbm.at[gidx], rows, sem).start()

        def rmw(pkv, gidx, rows, sem):
            pltpu.make_async_copy(p_hbm.at[gidx], rows, sem).wait()

            def group_rmw(g, carry):
                ldv = (pkv[pl.ds(g * 16, 16)] & (512 - 1)) * DW
                for j in range(16):
                    base = ldv[j]
                    k = g * 16 + j
                    a = accs[j % 3]
                    for c4 in range(DW // 16):
                        sl = pl.ds(base + c4 * 16, 16)
                        va = plsc.bitcast(rows[k, pl.ds(c4 * 16, 16)],
                                          jnp.bfloat16)
                        vo = plsc.bitcast(a[sl], jnp.bfloat16)
                        a[sl] = plsc.bitcast(jnp.maximum(vo, va), jnp.int32)
                return carry

            lax.fori_loop(0, K // 16, group_rmw, 0)

        @pl.when(nb > 0)
        def _():
            start(0, *bufs[0])

        def pair_body(i, carry):
            b0 = 2 * i

            @pl.when(b0 + 1 < nb)
            def _():
                start(b0 + 1, *bufs[1])

            rmw(*bufs[0])

            @pl.when(b0 + 2 < nb)
            def _():
                start(b0 + 2, *bufs[0])

            @pl.when(b0 + 1 < nb)
            def _():
                rmw(*bufs[1])

            return carry

        lax.fori_loop(0, (nb + 1) // 2, pair_body, 0)

        def merge_body(i):
            sl = pl.ds(i * 16, 16)
            m0 = jnp.maximum(plsc.bitcast(acc0[sl], jnp.bfloat16),
                             plsc.bitcast(acc1[sl], jnp.bfloat16))
            m1 = plsc.bitcast(acc2[sl], jnp.bfloat16)
            acc0[sl] = plsc.bitcast(jnp.maximum(m0, m1), jnp.int32)

        plsc.parallel_loop(0, RPT * DW // 16, unroll=2)(merge_body)

        @pl.when(wid < NW - 1)
        def _():
            pltpu.sync_copy(acc0.at[pl.ds(0, RPT * DW)],
                            out_hbm.at[pl.ds(lo * DW, RPT * DW)])

        @pl.when(wid == NW - 1)
        def _():
            pltpu.sync_copy(acc0.at[pl.ds(0, LAST_ROWS * DW)],
                            out_hbm.at[pl.ds(lo * DW, LAST_ROWS * DW)])

    return seg_kernel(p, opk, cnt)


EPT = 2 * E // NW
KP = 160
NPCHUNKS = EPT // KP


def _edge_dots_sc(h, src_all, dst_all):
    mesh = plsc.VectorSubcoreMesh(core_axis_name="c", subcore_axis_name="s")

    @functools.partial(
        pl.kernel,
        out_type=jax.ShapeDtypeStruct((2 * E,), jnp.float32),
        mesh=mesh,
        compiler_params=pltpu.CompilerParams(needs_layout_passes=False,
                                             use_tc_tiling_on_sc=False),
        scratch_types=[
            pltpu.VMEM_SHARED((N_NODES, DW), jnp.int32),
            pltpu.VMEM((KP,), jnp.int32),
            pltpu.VMEM((KP,), jnp.int32),
            pltpu.VMEM((KP,), jnp.int32),
            pltpu.VMEM((KP,), jnp.int32),
            pltpu.VMEM((KP, DW), jnp.int32),
            pltpu.VMEM((KP, DW), jnp.int32),
            pltpu.VMEM((KP, DW), jnp.int32),
            pltpu.VMEM((KP, DW), jnp.int32),
            pltpu.VMEM((KP,), jnp.float32),
            pltpu.SemaphoreType.DMA,
            pltpu.SemaphoreType.DMA,
        ],
    )
    def dot_kernel(h_hbm, src_hbm, dst_hbm, out_hbm, hsh,
                   sa0, sb0, sa1, sb1, ra0, rb0, ra1, rb1, scv, sem0, sem1):
        wid = lax.axis_index("s") * NC + lax.axis_index("c")
        wbase = wid * EPT
        lane = lax.iota(jnp.int32, 16)

        @pl.when(lax.axis_index("s") == 0)
        def _():
            pltpu.sync_copy(h_hbm, hsh)

        plsc.subcore_barrier()

        def start(c, sa, sb, ra, rb, sem):
            base = pl.multiple_of(wbase + c * KP, 8)
            pltpu.sync_copy(src_hbm.at[pl.ds(base, KP)], sa)
            pltpu.sync_copy(dst_hbm.at[pl.ds(base, KP)], sb)
            pltpu.make_async_copy(hsh.at[sa], ra, sem).start()
            pltpu.make_async_copy(hsh.at[sb], rb, sem).start()

        def compute(c, sa, sb, ra, rb, sem):
            pltpu.make_async_copy(hsh.at[sa], ra, sem).wait()
            pltpu.make_async_copy(hsh.at[sb], rb, sem).wait()

            def per_group(g):
                vec = jnp.zeros((16,), jnp.float32)
                for j in range(16):
                    k = g * 16 + j
                    accv = jnp.zeros((16,), jnp.float32)
                    for c4 in range(DW // 16):
                        sl = pl.ds(c4 * 16, 16)
                        va = plsc.bitcast(ra[k, sl], jnp.bfloat16)
                        vb = plsc.bitcast(rb[k, sl], jnp.bfloat16)
                        prod = va * vb
                        p0, p1 = plsc.unpack(
                            prod, format=plsc.PackFormat.INTERLEAVED)
                        accv = accv + p0 + p1
                    vec = jnp.where(lane == j, jnp.sum(accv), vec)
                scv[pl.ds(g * 16, 16)] = vec

            plsc.parallel_loop(0, KP // 16)(per_group)
            base = pl.multiple_of(wbase + c * KP, 8)
            pltpu.sync_copy(scv, out_hbm.at[pl.ds(base, KP)])

        bufs = ((sa0, sb0, ra0, rb0, sem0), (sa1, sb1, ra1, rb1, sem1))

        start(0, *bufs[0])

        def pair_body(i, carry):
            c0 = 2 * i

            @pl.when(c0 + 1 < NPCHUNKS)
            def _():
                start(c0 + 1, *bufs[1])

            compute(c0, *bufs[0])

            @pl.when(c0 + 2 < NPCHUNKS)
            def _():
                start(c0 + 2, *bufs[0])

            @pl.when(c0 + 1 < NPCHUNKS)
            def _():
                compute(c0 + 1, *bufs[1])

            return carry

        lax.fori_loop(0, (NPCHUNKS + 1) // 2, pair_body, 0)

    return dot_kernel(h, src_all, dst_all)


def _b16_as_i32(t):
    return lax.bitcast_convert_type(t.reshape(N_NODES, DW, 2), jnp.int32)


def _i32_as_b16(t):
    return lax.bitcast_convert_type(
        t.reshape(N_NODES, DW), jnp.bfloat16).reshape(N_NODES, D)


def kernel(x, edge_index, neg_edge_index, Wp1, bp1, Ws1, Wn1, b1,
           Wp2, bp2, Ws2, Wn2, b2):
    src, dst = edge_index[0], edge_index[1]
    opk, cnt = _partition_sc(src, dst)
    p1 = _b16_as_i32(_dense_pool(x, Wp1, bp1))
    n1 = _i32_as_b16(_segmax_sc(p1, opk, cnt))
    h1 = _dense2(x, Ws1, n1, Wn1, b1, relu=True, out_bf16=False)
    p2 = _b16_as_i32(_dense_pool(h1, Wp2, bp2))
    n2 = _i32_as_b16(_segmax_sc(p2, opk, cnt))
    h2 = _b16_as_i32(_dense2(h1, Ws2, n2, Wn2, b2, relu=False, out_bf16=True))
    src_all = jnp.concatenate([src, neg_edge_index[0]])
    dst_all = jnp.concatenate([dst, neg_edge_index[1]])
    scores = _edge_dots_sc(h2, src_all, dst_all)
    pos = scores[:E].reshape(E, 1)
    neg = scores[E:].reshape(E, 1)
    return (pos, neg)


# speedup vs baseline: 6.0733x; 1.3098x over previous
import functools

import jax
import jax.numpy as jnp
from jax import lax
from jax.experimental import pallas as pl
from jax.experimental.pallas import tpu as pltpu
from jax.experimental.pallas import tpu_sc as plsc

N_NODES = 10000
E = 320000
D = 128
DW = D // 2
BN = 1000

NC = 2
NS = 16
NW = NC * NS
RPT = 313
LAST_ROWS = N_NODES - (NW - 1) * RPT
TRASH = RPT
CHUNK = 2560
NCHUNKS = E // CHUNK
NGROUPS = CHUNK // 16
K = 256
FB = 4096
OB = 2 * FB + 16
LCAP = E + OB - 16 + 2048
LCAP = ((LCAP + 2047) // 2048) * 2048


def _dense_pool(x, W, b):
    def body(x_ref, w_ref, b_ref, o_ref):
        y = jnp.dot(x_ref[...], w_ref[...], preferred_element_type=jnp.float32)
        y = jnp.maximum(y + b_ref[...], 0.0)
        o_ref[...] = y.astype(jnp.bfloat16)

    n = x.shape[0]
    return pl.pallas_call(
        body,
        grid=(n // BN,),
        in_specs=[
            pl.BlockSpec((BN, D), lambda i: (i, 0)),
            pl.BlockSpec((D, D), lambda i: (0, 0)),
            pl.BlockSpec((1, D), lambda i: (0, 0)),
        ],
        out_specs=pl.BlockSpec((BN, D), lambda i: (i, 0)),
        out_shape=jax.ShapeDtypeStruct((n, D), jnp.bfloat16),
    )(x, W, b.reshape(1, D))


def _dense2(x, Wa, n_agg, Wb, b, relu, out_bf16):
    def body(x_ref, wa_ref, n_ref, wb_ref, b_ref, o_ref):
        y = jnp.dot(x_ref[...], wa_ref[...], preferred_element_type=jnp.float32)
        nv = n_ref[...].astype(jnp.float32)
        y = y + jnp.dot(nv, wb_ref[...], preferred_element_type=jnp.float32)
        y = y + b_ref[...]
        if relu:
            y = jnp.maximum(y, 0.0)
        o_ref[...] = y.astype(o_ref.dtype)

    n = x.shape[0]
    odtype = jnp.bfloat16 if out_bf16 else jnp.float32
    return pl.pallas_call(
        body,
        grid=(n // BN,),
        in_specs=[
            pl.BlockSpec((BN, D), lambda i: (i, 0)),
            pl.BlockSpec((D, D), lambda i: (0, 0)),
            pl.BlockSpec((BN, D), lambda i: (i, 0)),
            pl.BlockSpec((D, D), lambda i: (0, 0)),
            pl.BlockSpec((1, D), lambda i: (0, 0)),
        ],
        out_specs=pl.BlockSpec((BN, D), lambda i: (i, 0)),
        out_shape=jax.ShapeDtypeStruct((n, D), odtype),
    )(x, Wa, n_agg, Wb, b.reshape(1, D))


def _partition_sc(src, dst):
    mesh = plsc.VectorSubcoreMesh(core_axis_name="c", subcore_axis_name="s")

    @functools.partial(
        pl.kernel,
        out_type=(
            jax.ShapeDtypeStruct((NW * LCAP,), jnp.int32),
            jax.ShapeDtypeStruct((NW * 16,), jnp.int32),
        ),
        mesh=mesh,
        compiler_params=pltpu.CompilerParams(needs_layout_passes=False),
        scratch_types=[
            pltpu.VMEM((CHUNK,), jnp.int32),
            pltpu.VMEM((CHUNK,), jnp.int32),
            pltpu.VMEM((OB,), jnp.int32),
            pltpu.VMEM((16,), jnp.int32),
        ],
    )
    def part_kernel(src_hbm, dst_hbm, opk_hbm, cnt_hbm, srcv, dstv, obuf, cntv):
        wid = lax.axis_index("s") * NC + lax.axis_index("c")
        lo = wid * RPT
        wbase = wid * LCAP

        def chunk_body(ci, carry):
            nacc, written = carry
            ebase = ci * CHUNK
            pltpu.sync_copy(src_hbm.at[pl.ds(ebase, CHUNK)], srcv)
            pltpu.sync_copy(dst_hbm.at[pl.ds(ebase, CHUNK)], dstv)

            def group_body(g, nacc):
                off = g * 16
                d16 = dstv[pl.ds(off, 16)]
                s16 = srcv[pl.ds(off, 16)]
                l16 = d16 - lo
                m = (l16 >= 0) & (l16 < RPT)
                packed = jnp.bitwise_or(jnp.left_shift(s16, 9), l16)
                plsc.store_compressed(obuf.at[pl.ds(nacc, 16)], packed, mask=m)
                return nacc + plsc.all_reduce_population_count(m)[0]

            nacc = plsc.parallel_loop(0, NGROUPS, unroll=4,
                                      carry=nacc)(group_body)

            full = nacc >= FB

            @pl.when(full)
            def _():
                off = pl.multiple_of(wbase + written, 8)
                pltpu.sync_copy(obuf.at[pl.ds(0, FB)],
                                opk_hbm.at[pl.ds(off, FB)])
                for g in range((OB - FB) // 16):
                    obuf[pl.ds(g * 16, 16)] = obuf[pl.ds(FB + g * 16, 16)]

            nacc = jnp.where(full, nacc - FB, nacc)
            written = jnp.where(full, written + FB, written)
            return (nacc, written)

        nacc, written = lax.fori_loop(
            0, NCHUNKS, chunk_body, (jnp.int32(0), jnp.int32(0)))

        lane = lax.iota(jnp.int32, 16)
        trash = jnp.full((16,), TRASH, jnp.int32)

        def pad_body(g, carry):
            off = g * 16
            keep = (lane + off) < nacc
            obuf[pl.ds(off, 16)] = jnp.where(keep, obuf[pl.ds(off, 16)], trash)
            return carry

        lax.fori_loop(0, OB // 16, pad_body, 0)

        @pl.when(nacc > 0)
        def _():
            off = pl.multiple_of(wbase + written, 8)
            pltpu.sync_copy(obuf.at[pl.ds(0, FB)],
                            opk_hbm.at[pl.ds(off, FB)])

        pcount = ((nacc + K - 1) // K) * K
        cntv[pl.ds(0, 16)] = jnp.zeros((16,), jnp.int32) + (written + pcount)
        pltpu.sync_copy(cntv, cnt_hbm.at[pl.ds(wid * 16, 16)])

    return part_kernel(src, dst)


def _segmax_sc(p, opk, cnt):
    mesh = plsc.VectorSubcoreMesh(core_axis_name="c", subcore_axis_name="s")

    @functools.partial(
        pl.kernel,
        out_type=jax.ShapeDtypeStruct((N_NODES * DW,), jnp.int32),
        mesh=mesh,
        compiler_params=pltpu.CompilerParams(needs_layout_passes=False,
                                             use_tc_tiling_on_sc=False),
        scratch_types=[
            pltpu.VMEM((16,), jnp.int32),
            pltpu.VMEM((K + 16,), jnp.int32),
            pltpu.VMEM((K + 16,), jnp.int32),
            pltpu.VMEM((K,), jnp.int32),
            pltpu.VMEM((K,), jnp.int32),
            pltpu.VMEM((K, DW), jnp.int32),
            pltpu.VMEM((K, DW), jnp.int32),
            pltpu.VMEM(((RPT + 1) * DW,), jnp.int32),
            pltpu.VMEM(((RPT + 1) * DW,), jnp.int32),
            pltpu.VMEM(((RPT + 1) * DW,), jnp.int32),
            pltpu.SemaphoreType.DMA,
            pltpu.SemaphoreType.DMA,
        ],
    )
    def seg_kernel(p_hbm, opk_hbm, cnt_hbm, out_hbm,
                   cntv, pkv0, pkv1, gidx0, gidx1, rows0, rows1,
                   acc0, acc1, acc2, sem0, sem1):
        wid = lax.axis_index("s") * NC + lax.axis_index("c")
        lo = wid * RPT
        wbase = wid * LCAP
        accs = (acc0, acc1, acc2)

        def zero_body(i):
            z = jnp.zeros((16,), jnp.int32)
            for a in accs:
                a[pl.ds(i * 16, 16)] = z

        plsc.parallel_loop(0, (RPT + 1) * DW // 16, unroll=2)(zero_body)

        pltpu.sync_copy(cnt_hbm.at[pl.ds(wid * 16, 16)], cntv)
        nb = cntv[pl.ds(0, 16)][0] // K

        bufs = ((pkv0, gidx0, rows0, sem0), (pkv1, gidx1, rows1, sem1))

        def start(b, pkv, gidx, rows, sem):
            boff = pl.multiple_of(wbase + b * K, 8)
            pltpu.sync_copy(opk_hbm.at[pl.ds(boff, K)], pkv.at[pl.ds(0, K)])
            for g in range(K // 16):
                gidx[pl.ds(g * 16, 16)] = lax.shift_right_logical(
                    pkv[pl.ds(g * 16, 16)], 9)
            pltpu.make_async_copy(p_hbm.at[gidx], rows, sem).start()

        def rmw(pkv, gidx, rows, sem):
            pltpu.make_async_copy(p_hbm.at[gidx], rows, sem).wait()

            def group_rmw(g, carry):
                ldv = (pkv[pl.ds(g * 16, 16)] & (512 - 1)) * DW
                for j in range(16):
                    base = ldv[j]
                    k = g * 16 + j
                    a = accs[j % 3]
                    for c4 in range(DW // 16):
                        sl = pl.ds(base + c4 * 16, 16)
                        va = plsc.bitcast(rows[k, pl.ds(c4 * 16, 16)],
                                          jnp.bfloat16)
                        vo = plsc.bitcast(a[sl], jnp.bfloat16)
                        a[sl] = plsc.bitcast(jnp.maximum(vo, va), jnp.int32)
                return carry

            lax.fori_loop(0, K // 16, group_rmw, 0)

        @pl.when(nb > 0)
        def _():
            start(0, *bufs[0])

        def pair_body(i, carry):
            b0 = 2 * i

            @pl.when(b0 + 1 < nb)
            def _():
                start(b0 + 1, *bufs[1])

            rmw(*bufs[0])

            @pl.when(b0 + 2 < nb)
            def _():
                start(b0 + 2, *bufs[0])

            @pl.when(b0 + 1 < nb)
            def _():
                rmw(*bufs[1])

            return carry

        lax.fori_loop(0, (nb + 1) // 2, pair_body, 0)

        def merge_body(i):
            sl = pl.ds(i * 16, 16)
            m0 = jnp.maximum(plsc.bitcast(acc0[sl], jnp.bfloat16),
                             plsc.bitcast(acc1[sl], jnp.bfloat16))
            m1 = plsc.bitcast(acc2[sl], jnp.bfloat16)
            acc0[sl] = plsc.bitcast(jnp.maximum(m0, m1), jnp.int32)

        plsc.parallel_loop(0, RPT * DW // 16, unroll=2)(merge_body)

        @pl.when(wid < NW - 1)
        def _():
            pltpu.sync_copy(acc0.at[pl.ds(0, RPT * DW)],
                            out_hbm.at[pl.ds(lo * DW, RPT * DW)])

        @pl.when(wid == NW - 1)
        def _():
            pltpu.sync_copy(acc0.at[pl.ds(0, LAST_ROWS * DW)],
                            out_hbm.at[pl.ds(lo * DW, LAST_ROWS * DW)])

    return seg_kernel(p, opk, cnt)


EPT = 2 * E // NW
KP = 400
NPCHUNKS = EPT // KP


def _edge_dots_sc(h, src_all, dst_all):
    mesh = plsc.VectorSubcoreMesh(core_axis_name="c", subcore_axis_name="s")

    @functools.partial(
        pl.kernel,
        out_type=jax.ShapeDtypeStruct((2 * E,), jnp.float32),
        mesh=mesh,
        compiler_params=pltpu.CompilerParams(needs_layout_passes=False,
                                             use_tc_tiling_on_sc=False),
        scratch_types=[
            pltpu.VMEM((KP,), jnp.int32),
            pltpu.VMEM((KP,), jnp.int32),
            pltpu.VMEM((KP,), jnp.int32),
            pltpu.VMEM((KP,), jnp.int32),
            pltpu.VMEM((KP, DW), jnp.int32),
            pltpu.VMEM((KP, DW), jnp.int32),
            pltpu.VMEM((KP, DW), jnp.int32),
            pltpu.VMEM((KP, DW), jnp.int32),
            pltpu.VMEM((KP,), jnp.float32),
            pltpu.SemaphoreType.DMA,
            pltpu.SemaphoreType.DMA,
        ],
    )
    def dot_kernel(h_hbm, src_hbm, dst_hbm, out_hbm,
                   sa0, sb0, sa1, sb1, ra0, rb0, ra1, rb1, scv, sem0, sem1):
        wid = lax.axis_index("s") * NC + lax.axis_index("c")
        wbase = wid * EPT
        lane = lax.iota(jnp.int32, 16)

        def start(c, sa, sb, ra, rb, sem):
            base = pl.multiple_of(wbase + c * KP, 8)
            pltpu.sync_copy(src_hbm.at[pl.ds(base, KP)], sa)
            pltpu.sync_copy(dst_hbm.at[pl.ds(base, KP)], sb)
            pltpu.make_async_copy(h_hbm.at[sa], ra, sem).start()
            pltpu.make_async_copy(h_hbm.at[sb], rb, sem).start()

        def compute(c, sa, sb, ra, rb, sem):
            pltpu.make_async_copy(h_hbm.at[sa], ra, sem).wait()
            pltpu.make_async_copy(h_hbm.at[sb], rb, sem).wait()

            def per_group(g):
                parts = []
                for j in range(16):
                    k = g * 16 + j
                    accv = jnp.zeros((16,), jnp.float32)
                    for c4 in range(DW // 16):
                        sl = pl.ds(c4 * 16, 16)
                        va = plsc.bitcast(ra[k, sl], jnp.bfloat16)
                        vb = plsc.bitcast(rb[k, sl], jnp.bfloat16)
                        prod = va * vb
                        p0, p1 = plsc.unpack(
                            prod, format=plsc.PackFormat.INTERLEAVED)
                        accv = accv + p0 + p1
                    parts.append(jnp.where(lane == j, jnp.sum(accv), 0.0))
                while len(parts) > 1:
                    parts = [a + b for a, b in zip(parts[::2], parts[1::2])]
                scv[pl.ds(g * 16, 16)] = parts[0]

            plsc.parallel_loop(0, KP // 16)(per_group)
            base = pl.multiple_of(wbase + c * KP, 8)
            pltpu.sync_copy(scv, out_hbm.at[pl.ds(base, KP)])

        bufs = ((sa0, sb0, ra0, rb0, sem0), (sa1, sb1, ra1, rb1, sem1))

        start(0, *bufs[0])

        def pair_body(i, carry):
            c0 = 2 * i

            @pl.when(c0 + 1 < NPCHUNKS)
            def _():
                start(c0 + 1, *bufs[1])

            compute(c0, *bufs[0])

            @pl.when(c0 + 2 < NPCHUNKS)
            def _():
                start(c0 + 2, *bufs[0])

            @pl.when(c0 + 1 < NPCHUNKS)
            def _():
                compute(c0 + 1, *bufs[1])

            return carry

        lax.fori_loop(0, (NPCHUNKS + 1) // 2, pair_body, 0)

    return dot_kernel(h, src_all, dst_all)


def _b16_as_i32(t):
    return lax.bitcast_convert_type(t.reshape(N_NODES, DW, 2), jnp.int32)


def _i32_as_b16(t):
    return lax.bitcast_convert_type(
        t.reshape(N_NODES, DW), jnp.bfloat16).reshape(N_NODES, D)


def kernel(x, edge_index, neg_edge_index, Wp1, bp1, Ws1, Wn1, b1,
           Wp2, bp2, Ws2, Wn2, b2):
    src, dst = edge_index[0], edge_index[1]
    opk, cnt = _partition_sc(src, dst)
    p1 = _b16_as_i32(_dense_pool(x, Wp1, bp1))
    n1 = _i32_as_b16(_segmax_sc(p1, opk, cnt))
    h1 = _dense2(x, Ws1, n1, Wn1, b1, relu=True, out_bf16=False)
    p2 = _b16_as_i32(_dense_pool(h1, Wp2, bp2))
    n2 = _i32_as_b16(_segmax_sc(p2, opk, cnt))
    h2 = _b16_as_i32(_dense2(h1, Ws2, n2, Wn2, b2, relu=False, out_bf16=True))
    src_all = jnp.concatenate([src, neg_edge_index[0]])
    dst_all = jnp.concatenate([dst, neg_edge_index[1]])
    scores = _edge_dots_sc(h2, src_all, dst_all)
    pos = scores[:E].reshape(E, 1)
    neg = scores[E:].reshape(E, 1)
    return (pos, neg)


# speedup vs baseline: 6.1168x; 1.0072x over previous
import functools

import jax
import jax.numpy as jnp
from jax import lax
from jax.experimental import pallas as pl
from jax.experimental.pallas import tpu as pltpu
from jax.experimental.pallas import tpu_sc as plsc

N_NODES = 10000
E = 320000
D = 128
DW = D // 2
BN = 1000

NC = 2
NS = 16
NW = NC * NS
RPT = 313
LAST_ROWS = N_NODES - (NW - 1) * RPT
TRASH = RPT
CHUNK = 2560
NCHUNKS = E // CHUNK
NGROUPS = CHUNK // 16
K = 256
FB = 4096
OB = 2 * FB + 16
LCAP = E + OB - 16 + 2048
LCAP = ((LCAP + 2047) // 2048) * 2048


def _dense_pool(x, W, b):
    def body(x_ref, w_ref, b_ref, o_ref):
        y = jnp.dot(x_ref[...], w_ref[...], preferred_element_type=jnp.float32)
        y = jnp.maximum(y + b_ref[...], 0.0)
        o_ref[...] = y.astype(jnp.bfloat16)

    n = x.shape[0]
    return pl.pallas_call(
        body,
        grid=(n // BN,),
        in_specs=[
            pl.BlockSpec((BN, D), lambda i: (i, 0)),
            pl.BlockSpec((D, D), lambda i: (0, 0)),
            pl.BlockSpec((1, D), lambda i: (0, 0)),
        ],
        out_specs=pl.BlockSpec((BN, D), lambda i: (i, 0)),
        out_shape=jax.ShapeDtypeStruct((n, D), jnp.bfloat16),
    )(x, W, b.reshape(1, D))


def _dense2(x, Wa, n_agg, Wb, b, relu, out_bf16):
    def body(x_ref, wa_ref, n_ref, wb_ref, b_ref, o_ref):
        y = jnp.dot(x_ref[...], wa_ref[...], preferred_element_type=jnp.float32)
        nv = n_ref[...].astype(jnp.float32)
        y = y + jnp.dot(nv, wb_ref[...], preferred_element_type=jnp.float32)
        y = y + b_ref[...]
        if relu:
            y = jnp.maximum(y, 0.0)
        o_ref[...] = y.astype(o_ref.dtype)

    n = x.shape[0]
    odtype = jnp.bfloat16 if out_bf16 else jnp.float32
    return pl.pallas_call(
        body,
        grid=(n // BN,),
        in_specs=[
            pl.BlockSpec((BN, D), lambda i: (i, 0)),
            pl.BlockSpec((D, D), lambda i: (0, 0)),
            pl.BlockSpec((BN, D), lambda i: (i, 0)),
            pl.BlockSpec((D, D), lambda i: (0, 0)),
            pl.BlockSpec((1, D), lambda i: (0, 0)),
        ],
        out_specs=pl.BlockSpec((BN, D), lambda i: (i, 0)),
        out_shape=jax.ShapeDtypeStruct((n, D), odtype),
    )(x, Wa, n_agg, Wb, b.reshape(1, D))


def _dense2_pool(x, Wa, n_agg, Wb, b, Wp, bp):
    def body(x_ref, wa_ref, n_ref, wb_ref, b_ref, wp_ref, bp_ref,
             h_ref, p_ref):
        y = jnp.dot(x_ref[...], wa_ref[...], preferred_element_type=jnp.float32)
        nv = n_ref[...].astype(jnp.float32)
        y = y + jnp.dot(nv, wb_ref[...], preferred_element_type=jnp.float32)
        y = jnp.maximum(y + b_ref[...], 0.0)
        h_ref[...] = y
        p = jnp.dot(y, wp_ref[...], preferred_element_type=jnp.float32)
        p_ref[...] = jnp.maximum(p + bp_ref[...], 0.0).astype(jnp.bfloat16)

    n = x.shape[0]
    return pl.pallas_call(
        body,
        grid=(n // BN,),
        in_specs=[
            pl.BlockSpec((BN, D), lambda i: (i, 0)),
            pl.BlockSpec((D, D), lambda i: (0, 0)),
            pl.BlockSpec((BN, D), lambda i: (i, 0)),
            pl.BlockSpec((D, D), lambda i: (0, 0)),
            pl.BlockSpec((1, D), lambda i: (0, 0)),
            pl.BlockSpec((D, D), lambda i: (0, 0)),
            pl.BlockSpec((1, D), lambda i: (0, 0)),
        ],
        out_specs=[
            pl.BlockSpec((BN, D), lambda i: (i, 0)),
            pl.BlockSpec((BN, D), lambda i: (i, 0)),
        ],
        out_shape=[
            jax.ShapeDtypeStruct((n, D), jnp.float32),
            jax.ShapeDtypeStruct((n, D), jnp.bfloat16),
        ],
    )(x, Wa, n_agg, Wb, b.reshape(1, D), Wp, bp.reshape(1, D))


def _partition_sc(src, dst):
    mesh = plsc.VectorSubcoreMesh(core_axis_name="c", subcore_axis_name="s")

    @functools.partial(
        pl.kernel,
        out_type=(
            jax.ShapeDtypeStruct((NW * LCAP,), jnp.int32),
            jax.ShapeDtypeStruct((NW * 16,), jnp.int32),
        ),
        mesh=mesh,
        compiler_params=pltpu.CompilerParams(needs_layout_passes=False),
        scratch_types=[
            pltpu.VMEM((CHUNK,), jnp.int32),
            pltpu.VMEM((CHUNK,), jnp.int32),
            pltpu.VMEM((OB,), jnp.int32),
            pltpu.VMEM((16,), jnp.int32),
        ],
    )
    def part_kernel(src_hbm, dst_hbm, opk_hbm, cnt_hbm, srcv, dstv, obuf, cntv):
        wid = lax.axis_index("s") * NC + lax.axis_index("c")
        lo = wid * RPT
        wbase = wid * LCAP

        def chunk_body(ci, carry):
            nacc, written = carry
            ebase = ci * CHUNK
            pltpu.sync_copy(src_hbm.at[pl.ds(ebase, CHUNK)], srcv)
            pltpu.sync_copy(dst_hbm.at[pl.ds(ebase, CHUNK)], dstv)

            def group_body(g, nacc):
                off = g * 16
                d16 = dstv[pl.ds(off, 16)]
                s16 = srcv[pl.ds(off, 16)]
                l16 = d16 - lo
                m = (l16 >= 0) & (l16 < RPT)
                packed = jnp.bitwise_or(jnp.left_shift(s16, 9), l16)
                plsc.store_compressed(obuf.at[pl.ds(nacc, 16)], packed, mask=m)
                return nacc + plsc.all_reduce_population_count(m)[0]

            nacc = plsc.parallel_loop(0, NGROUPS, unroll=4,
                                      carry=nacc)(group_body)

            full = nacc >= FB

            @pl.when(full)
            def _():
                off = pl.multiple_of(wbase + written, 8)
                pltpu.sync_copy(obuf.at[pl.ds(0, FB)],
                                opk_hbm.at[pl.ds(off, FB)])
                for g in range((OB - FB) // 16):
                    obuf[pl.ds(g * 16, 16)] = obuf[pl.ds(FB + g * 16, 16)]

            nacc = jnp.where(full, nacc - FB, nacc)
            written = jnp.where(full, written + FB, written)
            return (nacc, written)

        nacc, written = lax.fori_loop(
            0, NCHUNKS, chunk_body, (jnp.int32(0), jnp.int32(0)))

        lane = lax.iota(jnp.int32, 16)
        trash = jnp.full((16,), TRASH, jnp.int32)

        def pad_body(g, carry):
            off = g * 16
            keep = (lane + off) < nacc
            obuf[pl.ds(off, 16)] = jnp.where(keep, obuf[pl.ds(off, 16)], trash)
            return carry

        lax.fori_loop(0, OB // 16, pad_body, 0)

        @pl.when(nacc > 0)
        def _():
            off = pl.multiple_of(wbase + written, 8)
            pltpu.sync_copy(obuf.at[pl.ds(0, FB)],
                            opk_hbm.at[pl.ds(off, FB)])

        pcount = ((nacc + K - 1) // K) * K
        cntv[pl.ds(0, 16)] = jnp.zeros((16,), jnp.int32) + (written + pcount)
        pltpu.sync_copy(cntv, cnt_hbm.at[pl.ds(wid * 16, 16)])

    return part_kernel(src, dst)


def _segmax_sc(p, opk, cnt):
    mesh = plsc.VectorSubcoreMesh(core_axis_name="c", subcore_axis_name="s")

    @functools.partial(
        pl.kernel,
        out_type=jax.ShapeDtypeStruct((N_NODES * DW,), jnp.int32),
        mesh=mesh,
        compiler_params=pltpu.CompilerParams(needs_layout_passes=False,
                                             use_tc_tiling_on_sc=False),
        scratch_types=[
            pltpu.VMEM((16,), jnp.int32),
            pltpu.VMEM((K + 16,), jnp.int32),
            pltpu.VMEM((K + 16,), jnp.int32),
            pltpu.VMEM((K,), jnp.int32),
            pltpu.VMEM((K,), jnp.int32),
            pltpu.VMEM((K, DW), jnp.int32),
            pltpu.VMEM((K, DW), jnp.int32),
            pltpu.VMEM(((RPT + 1) * DW,), jnp.int32),
            pltpu.VMEM(((RPT + 1) * DW,), jnp.int32),
            pltpu.VMEM(((RPT + 1) * DW,), jnp.int32),
            pltpu.SemaphoreType.DMA,
            pltpu.SemaphoreType.DMA,
        ],
    )
    def seg_kernel(p_hbm, opk_hbm, cnt_hbm, out_hbm,
                   cntv, pkv0, pkv1, gidx0, gidx1, rows0, rows1,
                   acc0, acc1, acc2, sem0, sem1):
        wid = lax.axis_index("s") * NC + lax.axis_index("c")
        lo = wid * RPT
        wbase = wid * LCAP
        accs = (acc0, acc1, acc2)

        def zero_body(i):
            z = jnp.zeros((16,), jnp.int32)
            for a in accs:
                a[pl.ds(i * 16, 16)] = z

        plsc.parallel_loop(0, (RPT + 1) * DW // 16, unroll=2)(zero_body)

        pltpu.sync_copy(cnt_hbm.at[pl.ds(wid * 16, 16)], cntv)
        nb = cntv[pl.ds(0, 16)][0] // K

        bufs = ((pkv0, gidx0, rows0, sem0), (pkv1, gidx1, rows1, sem1))

        def start(b, pkv, gidx, rows, sem):
            boff = pl.multiple_of(wbase + b * K, 8)
            pltpu.sync_copy(opk_hbm.at[pl.ds(boff, K)], pkv.at[pl.ds(0, K)])
            for g in range(K // 16):
                gidx[pl.ds(g * 16, 16)] = lax.shift_right_logical(
                    pkv[pl.ds(g * 16, 16)], 9)
            pltpu.make_async_copy(p_hbm.at[gidx], rows, sem).start()

        def rmw(pkv, gidx, rows, sem):
            pltpu.make_async_copy(p_hbm.at[gidx], rows, sem).wait()

            def group_rmw(g, carry):
                ldv = (pkv[pl.ds(g * 16, 16)] & (512 - 1)) * DW
                for j in range(16):
                    base = ldv[j]
                    k = g * 16 + j
                    a = accs[j % 3]
                    for c4 in range(DW // 16):
                        sl = pl.ds(base + c4 * 16, 16)
                        va = plsc.bitcast(rows[k, pl.ds(c4 * 16, 16)],
                                          jnp.bfloat16)
                        vo = plsc.bitcast(a[sl], jnp.bfloat16)
                        a[sl] = plsc.bitcast(jnp.maximum(vo, va), jnp.int32)
                return carry

            lax.fori_loop(0, K // 16, group_rmw, 0)

        @pl.when(nb > 0)
        def _():
            start(0, *bufs[0])

        def pair_body(i, carry):
            b0 = 2 * i

            @pl.when(b0 + 1 < nb)
            def _():
                start(b0 + 1, *bufs[1])

            rmw(*bufs[0])

            @pl.when(b0 + 2 < nb)
            def _():
                start(b0 + 2, *bufs[0])

            @pl.when(b0 + 1 < nb)
            def _():
                rmw(*bufs[1])

            return carry

        lax.fori_loop(0, (nb + 1) // 2, pair_body, 0)

        def merge_body(i):
            sl = pl.ds(i * 16, 16)
            m0 = jnp.maximum(plsc.bitcast(acc0[sl], jnp.bfloat16),
                             plsc.bitcast(acc1[sl], jnp.bfloat16))
            m1 = plsc.bitcast(acc2[sl], jnp.bfloat16)
            acc0[sl] = plsc.bitcast(jnp.maximum(m0, m1), jnp.int32)

        plsc.parallel_loop(0, RPT * DW // 16, unroll=2)(merge_body)

        @pl.when(wid < NW - 1)
        def _():
            pltpu.sync_copy(acc0.at[pl.ds(0, RPT * DW)],
                            out_hbm.at[pl.ds(lo * DW, RPT * DW)])

        @pl.when(wid == NW - 1)
        def _():
            pltpu.sync_copy(acc0.at[pl.ds(0, LAST_ROWS * DW)],
                            out_hbm.at[pl.ds(lo * DW, LAST_ROWS * DW)])

    return seg_kernel(p, opk, cnt)


EPT = 2 * E // NW
KP = 400
NPCHUNKS = EPT // KP


def _edge_dots_sc(h, src_all, dst_all):
    mesh = plsc.VectorSubcoreMesh(core_axis_name="c", subcore_axis_name="s")

    @functools.partial(
        pl.kernel,
        out_type=jax.ShapeDtypeStruct((2 * E,), jnp.float32),
        mesh=mesh,
        compiler_params=pltpu.CompilerParams(needs_layout_passes=False,
                                             use_tc_tiling_on_sc=False),
        scratch_types=[
            pltpu.VMEM((KP,), jnp.int32),
            pltpu.VMEM((KP,), jnp.int32),
            pltpu.VMEM((KP,), jnp.int32),
            pltpu.VMEM((KP,), jnp.int32),
            pltpu.VMEM((KP, DW), jnp.int32),
            pltpu.VMEM((KP, DW), jnp.int32),
            pltpu.VMEM((KP, DW), jnp.int32),
            pltpu.VMEM((KP, DW), jnp.int32),
            pltpu.VMEM((KP,), jnp.float32),
            pltpu.SemaphoreType.DMA,
            pltpu.SemaphoreType.DMA,
        ],
    )
    def dot_kernel(h_hbm, src_hbm, dst_hbm, out_hbm,
                   sa0, sb0, sa1, sb1, ra0, rb0, ra1, rb1, scv, sem0, sem1):
        wid = lax.axis_index("s") * NC + lax.axis_index("c")
        wbase = wid * EPT
        lane = lax.iota(jnp.int32, 16)

        def start(c, sa, sb, ra, rb, sem):
            base = pl.multiple_of(wbase + c * KP, 8)
            pltpu.sync_copy(src_hbm.at[pl.ds(base, KP)], sa)
            pltpu.sync_copy(dst_hbm.at[pl.ds(base, KP)], sb)
            pltpu.make_async_copy(h_hbm.at[sa], ra, sem).start()
            pltpu.make_async_copy(h_hbm.at[sb], rb, sem).start()

        def compute(c, sa, sb, ra, rb, sem):
            pltpu.make_async_copy(h_hbm.at[sa], ra, sem).wait()
            pltpu.make_async_copy(h_hbm.at[sb], rb, sem).wait()

            def per_group(g):
                parts = []
                for j in range(16):
                    k = g * 16 + j
                    accv = jnp.zeros((16,), jnp.float32)
                    for c4 in range(DW // 16):
                        sl = pl.ds(c4 * 16, 16)
                        va = plsc.bitcast(ra[k, sl], jnp.bfloat16)
                        vb = plsc.bitcast(rb[k, sl], jnp.bfloat16)
                        prod = va * vb
                        p0, p1 = plsc.unpack(
                            prod, format=plsc.PackFormat.INTERLEAVED)
                        accv = accv + p0 + p1
                    parts.append(jnp.where(lane == j, jnp.sum(accv), 0.0))
                while len(parts) > 1:
                    parts = [a + b for a, b in zip(parts[::2], parts[1::2])]
                scv[pl.ds(g * 16, 16)] = parts[0]

            plsc.parallel_loop(0, KP // 16)(per_group)
            base = pl.multiple_of(wbase + c * KP, 8)
            pltpu.sync_copy(scv, out_hbm.at[pl.ds(base, KP)])

        bufs = ((sa0, sb0, ra0, rb0, sem0), (sa1, sb1, ra1, rb1, sem1))

        start(0, *bufs[0])

        def pair_body(i, carry):
            c0 = 2 * i

            @pl.when(c0 + 1 < NPCHUNKS)
            def _():
                start(c0 + 1, *bufs[1])

            compute(c0, *bufs[0])

            @pl.when(c0 + 2 < NPCHUNKS)
            def _():
                start(c0 + 2, *bufs[0])

            @pl.when(c0 + 1 < NPCHUNKS)
            def _():
                compute(c0 + 1, *bufs[1])

            return carry

        lax.fori_loop(0, (NPCHUNKS + 1) // 2, pair_body, 0)

    return dot_kernel(h, src_all, dst_all)


def _b16_as_i32(t):
    return lax.bitcast_convert_type(t.reshape(N_NODES, DW, 2), jnp.int32)


def _i32_as_b16(t):
    return lax.bitcast_convert_type(
        t.reshape(N_NODES, DW), jnp.bfloat16).reshape(N_NODES, D)


def kernel(x, edge_index, neg_edge_index, Wp1, bp1, Ws1, Wn1, b1,
           Wp2, bp2, Ws2, Wn2, b2):
    src, dst = edge_index[0], edge_index[1]
    opk, cnt = _partition_sc(src, dst)
    p1 = _b16_as_i32(_dense_pool(x, Wp1, bp1))
    n1 = _i32_as_b16(_segmax_sc(p1, opk, cnt))
    h1, p2b = _dense2_pool(x, Ws1, n1, Wn1, b1, Wp2, bp2)
    p2 = _b16_as_i32(p2b)
    n2 = _i32_as_b16(_segmax_sc(p2, opk, cnt))
    h2 = _b16_as_i32(_dense2(h1, Ws2, n2, Wn2, b2, relu=False, out_bf16=True))
    src_all = jnp.concatenate([src, neg_edge_index[0]])
    dst_all = jnp.concatenate([dst, neg_edge_index[1]])
    scores = _edge_dots_sc(h2, src_all, dst_all)
    pos = scores[:E].reshape(E, 1)
    neg = scores[E:].reshape(E, 1)
    return (pos, neg)
